# Initial kernel scaffold; baseline (speedup 1.0000x reference)
import jax
import jax.numpy as jnp
from jax.experimental import pallas as pl


def kernel(feat1, feat2, msk, samp_bias1, samp_bias2, edge_index, W_mp, W_attn, b_attn, q_attn, W_fc, b_fc, M_disc):
    raise NotImplementedError("write your pallas kernel here")


# speedup vs baseline: 5.0258x; 5.0258x over previous
import functools

import jax
import jax.numpy as jnp
from jax import lax
from jax.experimental import pallas as pl
from jax.experimental.pallas import tpu as pltpu
from jax.experimental.pallas import tpu_sc as plsc


def _build_sc_agg(N, D, E):
    NS = 16
    per_tile = E // NS
    CH = 128
    n_main, tail = divmod(per_tile, CH)
    rpt = ((N + NS - 1) // NS + CH - 1) // CH * CH
    last_rows = N - (NS - 1) * rpt
    assert 0 < last_rows <= rpt and last_rows % 16 == 0
    nz_full = rpt // CH
    nz_last, z_tail = divmod(last_rows, CH)
    assert z_tail % 16 == 0
    rb = 80
    assert rpt % rb == 0 and last_rows % rb == 0 and rb <= CH

    mesh = plsc.VectorSubcoreMesh(core_axis_name="c", subcore_axis_name="s")

    @functools.partial(
        pl.kernel,
        out_type=[
            jax.ShapeDtypeStruct((N, D), jnp.float32),
            jax.ShapeDtypeStruct((N, D), jnp.float32),
            jax.ShapeDtypeStruct((NS * N,), jnp.float32),
        ],
        mesh=mesh,
        scratch_types=[
            pltpu.VMEM((CH,), jnp.int32),
            pltpu.VMEM((CH,), jnp.int32),
            pltpu.VMEM((tail,), jnp.int32),
            pltpu.VMEM((tail,), jnp.int32),
            pltpu.VMEM((CH, D), jnp.float32),
            pltpu.VMEM((N + 16,), jnp.float32),
            pltpu.VMEM_SHARED((N, D), jnp.float32),
            pltpu.SemaphoreType.DMA,
        ],
    )
    def sc_agg(feat1, feat2, srcv, dstv, agg1, agg2, degp,
               sidx, didx, sidx_t, didx_t, rows, degloc, acc, sem):
        cid = lax.axis_index("c")
        sid = lax.axis_index("s")

        zero16 = jnp.zeros((16,), jnp.float32)
        one16 = jnp.ones((16,), jnp.float32)

        def fill_row(r, _):
            for g in range(D // 16):
                rows[r, pl.ds(g * 16, 16)] = zero16
            return 0

        lax.fori_loop(0, CH, fill_row, 0, unroll=False)

        def zero_deg(i, _):
            degloc[pl.ds(i * 16, 16)] = zero16
            return 0

        lax.fori_loop(0, (N + 16) // 16, zero_deg, 0, unroll=False)

        base_r = sid * rpt

        @pl.when(sid < NS - 1)
        def _():
            for k in range(nz_full):
                pltpu.sync_copy(rows, acc.at[pl.ds(base_r + k * CH, CH)])

        @pl.when(sid == NS - 1)
        def _():
            b = (NS - 1) * rpt
            for k in range(nz_last):
                pltpu.sync_copy(rows, acc.at[pl.ds(b + k * CH, CH)])
            if z_tail:
                pltpu.sync_copy(rows.at[pl.ds(0, z_tail)],
                                acc.at[pl.ds(b + nz_last * CH, z_tail)])

        plsc.subcore_barrier()

        tbase = sid * per_tile

        unit16 = jnp.where(lax.iota(jnp.int32, 16) == 0, 1.0, 0.0)

        def deg_count(idx_ref, n):
            for g in range(n // 16):
                idx16 = idx_ref[pl.ds(g * 16, 16)]
                for l in range(16):
                    d = idx16[l]
                    plsc.addupdate(degloc.at[pl.ds(d, 16)], unit16)

        def run_branch(feat, do_deg):
            def body(i, _):
                base = pl.multiple_of(tbase + i * CH, 8)
                pltpu.sync_copy(srcv.at[pl.ds(base, CH)], sidx)
                pltpu.sync_copy(dstv.at[pl.ds(base, CH)], didx)
                pltpu.async_copy(feat.at[sidx], rows, sem).wait()
                pltpu.sync_copy(rows, acc.at[didx], add=True)
                if do_deg:
                    deg_count(didx, CH)
                return 0

            lax.fori_loop(0, n_main, body, 0, unroll=False)
            if tail:
                base = tbase + n_main * CH
                pltpu.sync_copy(srcv.at[pl.ds(base, tail)], sidx_t)
                pltpu.sync_copy(dstv.at[pl.ds(base, tail)], didx_t)
                pltpu.async_copy(feat.at[sidx_t], rows.at[pl.ds(0, tail)],
                                 sem).wait()
                pltpu.sync_copy(rows.at[pl.ds(0, tail)], acc.at[didx_t],
                                add=True)
                if do_deg:
                    deg_count(didx_t, tail)

        @pl.when(cid == 0)
        def _():
            run_branch(feat1, True)
            pltpu.sync_copy(degloc.at[pl.ds(0, N)], degp.at[pl.ds(sid * N, N)])

        @pl.when(cid == 1)
        def _():
            run_branch(feat2, False)

        plsc.subcore_barrier()

        n_rb = jnp.where(sid == NS - 1, last_rows // rb, rpt // rb)

        def rb_body(k, _):
            r0 = pl.multiple_of(base_r + k * rb, 8)
            pltpu.sync_copy(acc.at[pl.ds(r0, rb)], rows.at[pl.ds(0, rb)])

            @pl.when(cid == 0)
            def _():
                pltpu.sync_copy(rows.at[pl.ds(0, rb)], agg1.at[pl.ds(r0, rb)])

            @pl.when(cid == 1)
            def _():
                pltpu.sync_copy(rows.at[pl.ds(0, rb)], agg2.at[pl.ds(r0, rb)])

            return 0

        lax.fori_loop(0, n_rb, rb_body, 0, unroll=False)

    return sc_agg


def _metapath_acts(n_mp, agg1_ref, agg2_ref, deg_ref, wmp_ref):
    ones16 = jnp.ones((16, 1), jnp.float32)
    deg = jnp.dot(deg_ref[:, :], ones16,
                  preferred_element_type=jnp.float32)
    recip = 1.0 / jnp.maximum(deg, 1.0)
    a1 = agg1_ref[:, :] * recip
    a2 = agg2_ref[:, :] * recip
    h1s = []
    h2s = []

    def elu(x):
        return jnp.where(x > 0, x, jnp.exp(jnp.minimum(x, 0.0)) - 1.0)

    for m in range(n_mp):
        wm = wmp_ref[m, :, :]
        h1s.append(elu(jnp.dot(a1, wm, preferred_element_type=jnp.float32)))
        h2s.append(elu(jnp.dot(a2, wm, preferred_element_type=jnp.float32)))
    return h1s, h2s


def _tc_p0_body(N, n_mp,
                agg1_ref, agg2_ref, deg_ref, msk_ref, wmp_ref, wa_ref, bq_ref,
                md_ref,
                w1_ref, w2_ref, vv_ref,
                s1_ref, s2_ref, cs_ref, ms_ref):
    i = pl.program_id(0)
    nb = pl.num_programs(0)
    h1s, h2s = _metapath_acts(n_mp, agg1_ref, agg2_ref, deg_ref, wmp_ref)

    @pl.when(i == 0)
    def _():
        s1_ref[:, :] = jnp.zeros_like(s1_ref)
        s2_ref[:, :] = jnp.zeros_like(s2_ref)
        cs_ref[:, :] = jnp.zeros_like(cs_ref)
        ms_ref[:, :] = jnp.zeros_like(ms_ref)

    wa = wa_ref[:, :]
    b_attn = bq_ref[0:1, :]
    q_attn = bq_ref[1:2, :]
    mskb = msk_ref[:, :]
    for m in range(n_mp):
        t1 = jnp.tanh(jnp.dot(h1s[m], wa, preferred_element_type=jnp.float32)
                      + b_attn) * q_attn
        s1_ref[pl.ds(m, 1), :] += jnp.sum(t1, axis=0, keepdims=True)
        t2 = jnp.tanh(jnp.dot(h2s[m], wa, preferred_element_type=jnp.float32)
                      + b_attn) * q_attn
        s2_ref[pl.ds(m, 1), :] += jnp.sum(t2, axis=0, keepdims=True)
        cs_ref[pl.ds(m, 1), :] += jnp.sum(h1s[m] * mskb, axis=0,
                                          keepdims=True)
    ms_ref[:, :] += jnp.broadcast_to(
        jnp.sum(mskb).reshape(1, 1), ms_ref.shape)

    @pl.when(i == nb - 1)
    def _():
        for sacc, wref in ((s1_ref, w1_ref), (s2_ref, w2_ref)):
            sc = jnp.sum(sacc[:, :], axis=1, keepdims=True) / N
            mx = jnp.max(sc, axis=0, keepdims=True)
            e = jnp.exp(sc - mx)
            w = e / jnp.sum(e, axis=0, keepdims=True)
            wref[:, :] = jnp.broadcast_to(w, wref.shape)
        crow = (jnp.sum(w1_ref[:, :] * cs_ref[:, :], axis=0, keepdims=True)
                / ms_ref[0:1, :])
        cvec = 1.0 / (1.0 + jnp.exp(-crow))
        vv_ref[:, :] = lax.dot_general(
            cvec, md_ref[:, :], (((1,), (1,)), ((), ())),
            preferred_element_type=jnp.float32)


def _tc_p1_body(n_mp,
                agg1_ref, agg2_ref, deg_ref, wmp_ref, w1_ref, w2_ref, vv_ref,
                wfc_ref, bfc_ref,
                preds_ref, sc1_ref, sc2_ref):
    h1s, h2s = _metapath_acts(n_mp, agg1_ref, agg2_ref, deg_ref, wmp_ref)
    h1 = h1s[0] * w1_ref[pl.ds(0, 1), :]
    h2 = h2s[0] * w2_ref[pl.ds(0, 1), :]
    for m in range(1, n_mp):
        h1 = h1 + h1s[m] * w1_ref[pl.ds(m, 1), :]
        h2 = h2 + h2s[m] * w2_ref[pl.ds(m, 1), :]
    preds_ref[:, :] = (jnp.dot(h1, wfc_ref[:, :],
                               preferred_element_type=jnp.float32)
                       + bfc_ref[:, :])
    s1v = jnp.sum(h1 * vv_ref[:, :], axis=1, keepdims=True)
    s2v = jnp.sum(h2 * vv_ref[:, :], axis=1, keepdims=True)
    sc1_ref[:, :] = jnp.broadcast_to(s1v, sc1_ref.shape)
    sc2_ref[:, :] = jnp.broadcast_to(s2v, sc2_ref.shape)


def _build_tc_dense(N, D, n_mp, n_cls, B=1000):
    nb = N // B
    im_rows = lambda i: (i, 0)
    im_fixed = lambda i: (0, 0)

    p0 = pl.pallas_call(
        functools.partial(_tc_p0_body, N, n_mp),
        grid=(nb,),
        in_specs=[
            pl.BlockSpec((B, D), im_rows),
            pl.BlockSpec((B, D), im_rows),
            pl.BlockSpec((B, 16), im_rows),
            pl.BlockSpec((B, 1), im_rows),
            pl.BlockSpec((n_mp, D, D), lambda i: (0, 0, 0)),
            pl.BlockSpec((D, D), im_fixed),
            pl.BlockSpec((2, D), im_fixed),
            pl.BlockSpec((D, D), im_fixed),
        ],
        out_specs=[
            pl.BlockSpec((n_mp, D), im_fixed),
            pl.BlockSpec((n_mp, D), im_fixed),
            pl.BlockSpec((1, D), im_fixed),
        ],
        out_shape=[
            jax.ShapeDtypeStruct((n_mp, D), jnp.float32),
            jax.ShapeDtypeStruct((n_mp, D), jnp.float32),
            jax.ShapeDtypeStruct((1, D), jnp.float32),
        ],
        scratch_shapes=[
            pltpu.VMEM((n_mp, D), jnp.float32),
            pltpu.VMEM((n_mp, D), jnp.float32),
            pltpu.VMEM((n_mp, D), jnp.float32),
            pltpu.VMEM((1, D), jnp.float32),
        ],
        compiler_params=pltpu.CompilerParams(
            dimension_semantics=("arbitrary",)),
    )
    p1 = pl.pallas_call(
        functools.partial(_tc_p1_body, n_mp),
        grid=(nb,),
        in_specs=[
            pl.BlockSpec((B, D), im_rows),
            pl.BlockSpec((B, D), im_rows),
            pl.BlockSpec((B, 16), im_rows),
            pl.BlockSpec((n_mp, D, D), lambda i: (0, 0, 0)),
            pl.BlockSpec((n_mp, D), im_fixed),
            pl.BlockSpec((n_mp, D), im_fixed),
            pl.BlockSpec((1, D), im_fixed),
            pl.BlockSpec((D, n_cls), im_fixed),
            pl.BlockSpec((1, n_cls), im_fixed),
        ],
        out_specs=[
            pl.BlockSpec((B, n_cls), im_rows),
            pl.BlockSpec((B, 8), im_rows),
            pl.BlockSpec((B, 8), im_rows),
        ],
        out_shape=[
            jax.ShapeDtypeStruct((N, n_cls), jnp.float32),
            jax.ShapeDtypeStruct((N, 8), jnp.float32),
            jax.ShapeDtypeStruct((N, 8), jnp.float32),
        ],
        compiler_params=pltpu.CompilerParams(
            dimension_semantics=("arbitrary",)),
    )
    return p0, p1


def kernel(feat1, feat2, msk, samp_bias1, samp_bias2, edge_index,
           W_mp, W_attn, b_attn, q_attn, W_fc, b_fc, M_disc):
    N, D = feat1.shape
    E = edge_index.shape[1]
    n_mp = W_mp.shape[0]
    n_cls = W_fc.shape[1]

    src = edge_index[0]
    dst = edge_index[1]

    sc_agg = _build_sc_agg(N, D, E)
    agg1, agg2, degp = sc_agg(feat1, feat2, src, dst)
    degp2 = degp.reshape(16, N).T

    mskc = msk.reshape(N, 1)
    bq = jnp.stack([b_attn, q_attn])
    p0, p1 = _build_tc_dense(N, D, n_mp, n_cls)
    w3, w3b, vv = p0(agg1, agg2, degp2, mskc, W_mp, W_attn, bq, M_disc)
    preds, sc1f, sc2f = p1(agg1, agg2, degp2, W_mp, w3, w3b, vv,
                           W_fc, b_fc.reshape(1, n_cls))

    weights = w3[:, 0]
    sc_1 = sc1f[:, 0][None, :] + samp_bias1
    sc_2 = sc2f[:, 0][None, :] + samp_bias2
    reg = jnp.concatenate([sc_1, sc_2], axis=1)
    return (preds, weights, reg)


# speedup vs baseline: 8.8127x; 1.7535x over previous
import functools

import jax
import jax.numpy as jnp
from jax import lax
from jax.experimental import pallas as pl
from jax.experimental.pallas import tpu as pltpu
from jax.experimental.pallas import tpu_sc as plsc


def _build_sc_agg(N, D, E):
    NS = 16
    per_tile = E // NS
    CH = 64
    n_main, tail = divmod(per_tile, CH)
    rpt = ((N + NS - 1) // NS + CH - 1) // CH * CH
    last_rows = N - (NS - 1) * rpt
    assert 0 < last_rows <= rpt and last_rows % 16 == 0
    nz_full = rpt // CH
    nz_last, z_tail = divmod(last_rows, CH)
    assert z_tail % 16 == 0
    rb = 40
    assert rpt % rb == 0 and last_rows % rb == 0 and rb <= CH

    mesh = plsc.VectorSubcoreMesh(core_axis_name="c", subcore_axis_name="s")

    @functools.partial(
        pl.kernel,
        out_type=[
            jax.ShapeDtypeStruct((N, D), jnp.float32),
            jax.ShapeDtypeStruct((N, D), jnp.float32),
            jax.ShapeDtypeStruct((NS * N,), jnp.float32),
        ],
        mesh=mesh,
        scratch_types=[
            [pltpu.VMEM((CH,), jnp.int32)] * 3,
            [pltpu.VMEM((CH,), jnp.int32)] * 3,
            pltpu.VMEM((tail,), jnp.int32),
            pltpu.VMEM((tail,), jnp.int32),
            [pltpu.VMEM((CH, D), jnp.float32)] * 3,
            pltpu.VMEM((N + 16,), jnp.float32),
            pltpu.VMEM_SHARED((N, D), jnp.float32),
            [pltpu.SemaphoreType.DMA] * 3,
            [pltpu.SemaphoreType.DMA] * 3,
            [pltpu.SemaphoreType.DMA] * 3,
            [pltpu.SemaphoreType.DMA] * 3,
            pltpu.SemaphoreType.DMA,
        ],
    )
    def sc_agg(feat1, feat2, srcv, dstv, agg1, agg2, degp,
               sidx_b, didx_b, sidx_t, didx_t, rows_b, degloc, acc,
               sem_is, sem_id, sem_g, sem_s, sem):
        rows = rows_b[0]
        cid = lax.axis_index("c")
        sid = lax.axis_index("s")

        zero16 = jnp.zeros((16,), jnp.float32)
        one16 = jnp.ones((16,), jnp.float32)

        def fill_row(r, _):
            for g in range(D // 16):
                rows[r, pl.ds(g * 16, 16)] = zero16
            return 0

        lax.fori_loop(0, CH, fill_row, 0, unroll=False)

        def zero_deg(i, _):
            degloc[pl.ds(i * 16, 16)] = zero16
            return 0

        lax.fori_loop(0, (N + 16) // 16, zero_deg, 0, unroll=False)

        base_r = sid * rpt

        @pl.when(sid < NS - 1)
        def _():
            for k in range(nz_full):
                pltpu.sync_copy(rows, acc.at[pl.ds(base_r + k * CH, CH)])

        @pl.when(sid == NS - 1)
        def _():
            b = (NS - 1) * rpt
            for k in range(nz_last):
                pltpu.sync_copy(rows, acc.at[pl.ds(b + k * CH, CH)])
            if z_tail:
                pltpu.sync_copy(rows.at[pl.ds(0, z_tail)],
                                acc.at[pl.ds(b + nz_last * CH, z_tail)])

        plsc.subcore_barrier()

        tbase = sid * per_tile

        unit16 = jnp.where(lax.iota(jnp.int32, 16) == 0, 1.0, 0.0)

        def deg_count(idx_ref, n):
            for g in range(n // 16):
                idx16 = idx_ref[pl.ds(g * 16, 16)]
                for l in range(16):
                    d = idx16[l]
                    plsc.addupdate(degloc.at[pl.ds(d, 16)], unit16)

        def run_branch(feat, do_deg):
            def idx_load(i, s):
                base = pl.multiple_of(tbase + i * CH, 8)
                pltpu.async_copy(srcv.at[pl.ds(base, CH)], sidx_b[s],
                                 sem_is[s])
                pltpu.async_copy(dstv.at[pl.ds(base, CH)], didx_b[s],
                                 sem_id[s])

            def idx_wait(s):
                pltpu.make_async_copy(srcv.at[pl.ds(0, CH)], sidx_b[s],
                                      sem_is[s]).wait()
                pltpu.make_async_copy(dstv.at[pl.ds(0, CH)], didx_b[s],
                                      sem_id[s]).wait()

            def gather_start(s):
                pltpu.async_copy(feat.at[sidx_b[s]], rows_b[s], sem_g[s])

            def gather_wait(s):
                pltpu.make_async_copy(feat.at[sidx_b[s]], rows_b[s],
                                      sem_g[s]).wait()

            def scatter_start(s):
                pltpu.async_copy(rows_b[s], acc.at[didx_b[s]], sem_s[s],
                                 add=True)

            def scatter_wait(s):
                pltpu.make_async_copy(rows_b[s], acc.at[didx_b[s]],
                                      sem_s[s]).wait()

            def step(i, s, s1, s2, first):
                if not first:
                    scatter_wait(s2)
                    idx_load(i + 2, s2)
                idx_wait(s1)
                gather_start(s1)
                gather_wait(s)
                scatter_start(s)
                if do_deg:
                    deg_count(didx_b[s], CH)

            idx_load(0, 0)
            idx_load(1, 1)
            idx_wait(0)
            gather_start(0)
            idx_load(2, 2)
            step(0, 0, 1, 2, True)

            n_loop = n_main - 3
            peel = n_loop % 3
            for j in range(peel):
                step(1 + j, (1 + j) % 3, (2 + j) % 3, j % 3, False)
            i_base = 1 + peel

            def outer(o, _):
                i0 = i_base + o * 3
                for k in range(3):
                    sk = (i_base + k) % 3
                    step(i0 + k, sk, (sk + 1) % 3, (sk + 2) % 3, False)
                return 0

            lax.fori_loop(0, n_loop // 3, outer, 0, unroll=False)

            sa = (n_main - 2) % 3
            sb = (n_main - 1) % 3
            sc_ = (n_main - 3) % 3
            scatter_wait(sc_)
            idx_wait(sb)
            gather_start(sb)
            gather_wait(sa)
            scatter_start(sa)
            if do_deg:
                deg_count(didx_b[sa], CH)
            gather_wait(sb)
            scatter_start(sb)
            if do_deg:
                deg_count(didx_b[sb], CH)

            if tail:
                base = tbase + n_main * CH
                pltpu.sync_copy(srcv.at[pl.ds(base, tail)], sidx_t)
                pltpu.sync_copy(dstv.at[pl.ds(base, tail)], didx_t)
                pltpu.async_copy(feat.at[sidx_t], rows_b[sc_].at[pl.ds(0, tail)],
                                 sem).wait()
                pltpu.sync_copy(rows_b[sc_].at[pl.ds(0, tail)],
                                acc.at[didx_t], add=True)
                if do_deg:
                    deg_count(didx_t, tail)
            scatter_wait(sa)
            scatter_wait(sb)

        @pl.when(cid == 0)
        def _():
            run_branch(feat1, True)
            pltpu.sync_copy(degloc.at[pl.ds(0, N)], degp.at[pl.ds(sid * N, N)])

        @pl.when(cid == 1)
        def _():
            run_branch(feat2, False)

        plsc.subcore_barrier()

        n_rb = jnp.where(sid == NS - 1, last_rows // rb, rpt // rb)

        def rb_body(k, _):
            r0 = pl.multiple_of(base_r + k * rb, 8)
            pltpu.sync_copy(acc.at[pl.ds(r0, rb)], rows.at[pl.ds(0, rb)])

            @pl.when(cid == 0)
            def _():
                pltpu.sync_copy(rows.at[pl.ds(0, rb)], agg1.at[pl.ds(r0, rb)])

            @pl.when(cid == 1)
            def _():
                pltpu.sync_copy(rows.at[pl.ds(0, rb)], agg2.at[pl.ds(r0, rb)])

            return 0

        lax.fori_loop(0, n_rb, rb_body, 0, unroll=False)

    return sc_agg


def _metapath_acts(n_mp, agg1_ref, agg2_ref, deg_ref, wmp_ref):
    ones16 = jnp.ones((16, 1), jnp.float32)
    deg = jnp.dot(deg_ref[:, :], ones16,
                  preferred_element_type=jnp.float32)
    recip = 1.0 / jnp.maximum(deg, 1.0)
    a1 = agg1_ref[:, :] * recip
    a2 = agg2_ref[:, :] * recip
    h1s = []
    h2s = []

    def elu(x):
        return jnp.where(x > 0, x, jnp.exp(jnp.minimum(x, 0.0)) - 1.0)

    for m in range(n_mp):
        wm = wmp_ref[m, :, :]
        h1s.append(elu(jnp.dot(a1, wm, preferred_element_type=jnp.float32)))
        h2s.append(elu(jnp.dot(a2, wm, preferred_element_type=jnp.float32)))
    return h1s, h2s


def _tc_p0_body(N, n_mp,
                agg1_ref, agg2_ref, deg_ref, msk_ref, wmp_ref, wa_ref, bq_ref,
                md_ref,
                w1_ref, w2_ref, vv_ref,
                s1_ref, s2_ref, cs_ref, ms_ref):
    i = pl.program_id(0)
    nb = pl.num_programs(0)
    h1s, h2s = _metapath_acts(n_mp, agg1_ref, agg2_ref, deg_ref, wmp_ref)

    @pl.when(i == 0)
    def _():
        s1_ref[:, :] = jnp.zeros_like(s1_ref)
        s2_ref[:, :] = jnp.zeros_like(s2_ref)
        cs_ref[:, :] = jnp.zeros_like(cs_ref)
        ms_ref[:, :] = jnp.zeros_like(ms_ref)

    wa = wa_ref[:, :]
    b_attn = bq_ref[0:1, :]
    q_attn = bq_ref[1:2, :]
    mskb = msk_ref[:, :]
    for m in range(n_mp):
        t1 = jnp.tanh(jnp.dot(h1s[m], wa, preferred_element_type=jnp.float32)
                      + b_attn) * q_attn
        s1_ref[pl.ds(m, 1), :] += jnp.sum(t1, axis=0, keepdims=True)
        t2 = jnp.tanh(jnp.dot(h2s[m], wa, preferred_element_type=jnp.float32)
                      + b_attn) * q_attn
        s2_ref[pl.ds(m, 1), :] += jnp.sum(t2, axis=0, keepdims=True)
        cs_ref[pl.ds(m, 1), :] += jnp.sum(h1s[m] * mskb, axis=0,
                                          keepdims=True)
    ms_ref[:, :] += jnp.broadcast_to(
        jnp.sum(mskb).reshape(1, 1), ms_ref.shape)

    @pl.when(i == nb - 1)
    def _():
        for sacc, wref in ((s1_ref, w1_ref), (s2_ref, w2_ref)):
            sc = jnp.sum(sacc[:, :], axis=1, keepdims=True) / N
            mx = jnp.max(sc, axis=0, keepdims=True)
            e = jnp.exp(sc - mx)
            w = e / jnp.sum(e, axis=0, keepdims=True)
            wref[:, :] = jnp.broadcast_to(w, wref.shape)
        crow = (jnp.sum(w1_ref[:, :] * cs_ref[:, :], axis=0, keepdims=True)
                / ms_ref[0:1, :])
        cvec = 1.0 / (1.0 + jnp.exp(-crow))
        vv_ref[:, :] = lax.dot_general(
            cvec, md_ref[:, :], (((1,), (1,)), ((), ())),
            preferred_element_type=jnp.float32)


def _tc_p1_body(n_mp,
                agg1_ref, agg2_ref, deg_ref, wmp_ref, w1_ref, w2_ref, vv_ref,
                wfc_ref, bfc_ref,
                preds_ref, sc1_ref, sc2_ref):
    h1s, h2s = _metapath_acts(n_mp, agg1_ref, agg2_ref, deg_ref, wmp_ref)
    h1 = h1s[0] * w1_ref[pl.ds(0, 1), :]
    h2 = h2s[0] * w2_ref[pl.ds(0, 1), :]
    for m in range(1, n_mp):
        h1 = h1 + h1s[m] * w1_ref[pl.ds(m, 1), :]
        h2 = h2 + h2s[m] * w2_ref[pl.ds(m, 1), :]
    preds_ref[:, :] = (jnp.dot(h1, wfc_ref[:, :],
                               preferred_element_type=jnp.float32)
                       + bfc_ref[:, :])
    s1v = jnp.sum(h1 * vv_ref[:, :], axis=1, keepdims=True)
    s2v = jnp.sum(h2 * vv_ref[:, :], axis=1, keepdims=True)
    sc1_ref[:, :] = jnp.broadcast_to(s1v, sc1_ref.shape)
    sc2_ref[:, :] = jnp.broadcast_to(s2v, sc2_ref.shape)


def _build_tc_dense(N, D, n_mp, n_cls, B=1000):
    nb = N // B
    im_rows = lambda i: (i, 0)
    im_fixed = lambda i: (0, 0)

    p0 = pl.pallas_call(
        functools.partial(_tc_p0_body, N, n_mp),
        grid=(nb,),
        in_specs=[
            pl.BlockSpec((B, D), im_rows),
            pl.BlockSpec((B, D), im_rows),
            pl.BlockSpec((B, 16), im_rows),
            pl.BlockSpec((B, 1), im_rows),
            pl.BlockSpec((n_mp, D, D), lambda i: (0, 0, 0)),
            pl.BlockSpec((D, D), im_fixed),
            pl.BlockSpec((2, D), im_fixed),
            pl.BlockSpec((D, D), im_fixed),
        ],
        out_specs=[
            pl.BlockSpec((n_mp, D), im_fixed),
            pl.BlockSpec((n_mp, D), im_fixed),
            pl.BlockSpec((1, D), im_fixed),
        ],
        out_shape=[
            jax.ShapeDtypeStruct((n_mp, D), jnp.float32),
            jax.ShapeDtypeStruct((n_mp, D), jnp.float32),
            jax.ShapeDtypeStruct((1, D), jnp.float32),
        ],
        scratch_shapes=[
            pltpu.VMEM((n_mp, D), jnp.float32),
            pltpu.VMEM((n_mp, D), jnp.float32),
            pltpu.VMEM((n_mp, D), jnp.float32),
            pltpu.VMEM((1, D), jnp.float32),
        ],
        compiler_params=pltpu.CompilerParams(
            dimension_semantics=("arbitrary",)),
    )
    p1 = pl.pallas_call(
        functools.partial(_tc_p1_body, n_mp),
        grid=(nb,),
        in_specs=[
            pl.BlockSpec((B, D), im_rows),
            pl.BlockSpec((B, D), im_rows),
            pl.BlockSpec((B, 16), im_rows),
            pl.BlockSpec((n_mp, D, D), lambda i: (0, 0, 0)),
            pl.BlockSpec((n_mp, D), im_fixed),
            pl.BlockSpec((n_mp, D), im_fixed),
            pl.BlockSpec((1, D), im_fixed),
            pl.BlockSpec((D, n_cls), im_fixed),
            pl.BlockSpec((1, n_cls), im_fixed),
        ],
        out_specs=[
            pl.BlockSpec((B, n_cls), im_rows),
            pl.BlockSpec((B, 8), im_rows),
            pl.BlockSpec((B, 8), im_rows),
        ],
        out_shape=[
            jax.ShapeDtypeStruct((N, n_cls), jnp.float32),
            jax.ShapeDtypeStruct((N, 8), jnp.float32),
            jax.ShapeDtypeStruct((N, 8), jnp.float32),
        ],
        compiler_params=pltpu.CompilerParams(
            dimension_semantics=("arbitrary",)),
    )
    return p0, p1


def kernel(feat1, feat2, msk, samp_bias1, samp_bias2, edge_index,
           W_mp, W_attn, b_attn, q_attn, W_fc, b_fc, M_disc):
    N, D = feat1.shape
    E = edge_index.shape[1]
    n_mp = W_mp.shape[0]
    n_cls = W_fc.shape[1]

    src = edge_index[0]
    dst = edge_index[1]

    sc_agg = _build_sc_agg(N, D, E)
    agg1, agg2, degp = sc_agg(feat1, feat2, src, dst)
    degp2 = degp.reshape(16, N).T

    mskc = msk.reshape(N, 1)
    bq = jnp.stack([b_attn, q_attn])
    p0, p1 = _build_tc_dense(N, D, n_mp, n_cls)
    w3, w3b, vv = p0(agg1, agg2, degp2, mskc, W_mp, W_attn, bq, M_disc)
    preds, sc1f, sc2f = p1(agg1, agg2, degp2, W_mp, w3, w3b, vv,
                           W_fc, b_fc.reshape(1, n_cls))

    weights = w3[:, 0]
    sc_1 = sc1f[:, 0][None, :] + samp_bias1
    sc_2 = sc2f[:, 0][None, :] + samp_bias2
    reg = jnp.concatenate([sc_1, sc_2], axis=1)
    return (preds, weights, reg)


# speedup vs baseline: 9.3747x; 1.0638x over previous
import functools

import jax
import jax.numpy as jnp
from jax import lax
from jax.experimental import pallas as pl
from jax.experimental.pallas import tpu as pltpu
from jax.experimental.pallas import tpu_sc as plsc


def _build_sc_agg(N, D, E):
    NS = 16
    per_tile = E // NS
    CH = 80
    n_main, tail = divmod(per_tile, CH)
    n_cut = n_main // 2
    rpt = ((N + NS - 1) // NS + CH - 1) // CH * CH
    last_rows = N - (NS - 1) * rpt
    assert 0 < last_rows <= rpt and last_rows % 16 == 0
    nz_full = rpt // CH
    nz_last, z_tail = divmod(last_rows, CH)
    assert z_tail % 16 == 0
    rb = 40
    assert rpt % rb == 0 and last_rows % rb == 0 and rb <= CH

    mesh = plsc.VectorSubcoreMesh(core_axis_name="c", subcore_axis_name="s")

    @functools.partial(
        pl.kernel,
        out_type=[
            jax.ShapeDtypeStruct((N, D), jnp.float32),
            jax.ShapeDtypeStruct((N, D), jnp.float32),
            jax.ShapeDtypeStruct((2 * NS * N,), jnp.float32),
        ],
        mesh=mesh,
        scratch_types=[
            [pltpu.VMEM((CH,), jnp.int32)] * 3,
            [pltpu.VMEM((CH,), jnp.int32)] * 3,
            pltpu.VMEM((tail,), jnp.int32),
            pltpu.VMEM((tail,), jnp.int32),
            [pltpu.VMEM((CH, D), jnp.float32)] * 3,
            pltpu.VMEM((N + 16,), jnp.float32),
            pltpu.VMEM_SHARED((N, D), jnp.float32),
            [pltpu.SemaphoreType.DMA] * 3,
            [pltpu.SemaphoreType.DMA] * 3,
            [pltpu.SemaphoreType.DMA] * 3,
            [pltpu.SemaphoreType.DMA] * 3,
            pltpu.SemaphoreType.DMA,
        ],
    )
    def sc_agg(feat1, feat2, srcv, dstv, agg1, agg2, degp,
               sidx_b, didx_b, sidx_t, didx_t, rows_b, degloc, acc,
               sem_is, sem_id, sem_g, sem_s, sem):
        rows = rows_b[0]
        cid = lax.axis_index("c")
        sid = lax.axis_index("s")

        zero16 = jnp.zeros((16,), jnp.float32)
        one16 = jnp.ones((16,), jnp.float32)

        def fill_row(r, _):
            for g in range(D // 16):
                rows[r, pl.ds(g * 16, 16)] = zero16
            return 0

        lax.fori_loop(0, CH, fill_row, 0, unroll=False)

        def zero_deg(i, _):
            degloc[pl.ds(i * 16, 16)] = zero16
            return 0

        lax.fori_loop(0, (N + 16) // 16, zero_deg, 0, unroll=False)

        base_r = sid * rpt

        @pl.when(sid < NS - 1)
        def _():
            for k in range(nz_full):
                pltpu.sync_copy(rows, acc.at[pl.ds(base_r + k * CH, CH)])

        @pl.when(sid == NS - 1)
        def _():
            b = (NS - 1) * rpt
            for k in range(nz_last):
                pltpu.sync_copy(rows, acc.at[pl.ds(b + k * CH, CH)])
            if z_tail:
                pltpu.sync_copy(rows.at[pl.ds(0, z_tail)],
                                acc.at[pl.ds(b + nz_last * CH, z_tail)])

        plsc.subcore_barrier()

        tbase = sid * per_tile

        unit16 = jnp.where(lax.iota(jnp.int32, 16) == 0, 1.0, 0.0)

        def deg_count(idx_ref, n):
            for g in range(n // 16):
                idx16 = idx_ref[pl.ds(g * 16, 16)]
                for l in range(16):
                    d = idx16[l]
                    plsc.addupdate(degloc.at[pl.ds(d, 16)], unit16)

        def run_branch(feat, deg_pred):
            def idx_load(i, s):
                base = pl.multiple_of(tbase + i * CH, 8)
                pltpu.async_copy(srcv.at[pl.ds(base, CH)], sidx_b[s],
                                 sem_is[s])
                pltpu.async_copy(dstv.at[pl.ds(base, CH)], didx_b[s],
                                 sem_id[s])

            def idx_wait(s):
                pltpu.make_async_copy(srcv.at[pl.ds(0, CH)], sidx_b[s],
                                      sem_is[s]).wait()
                pltpu.make_async_copy(dstv.at[pl.ds(0, CH)], didx_b[s],
                                      sem_id[s]).wait()

            def gather_start(s):
                pltpu.async_copy(feat.at[sidx_b[s]], rows_b[s], sem_g[s])

            def gather_wait(s):
                pltpu.make_async_copy(feat.at[sidx_b[s]], rows_b[s],
                                      sem_g[s]).wait()

            def scatter_start(s):
                pltpu.async_copy(rows_b[s], acc.at[didx_b[s]], sem_s[s],
                                 add=True)

            def scatter_wait(s):
                pltpu.make_async_copy(rows_b[s], acc.at[didx_b[s]],
                                      sem_s[s]).wait()

            def maybe_deg(i, idx_ref, n):
                dp = deg_pred(i)
                if isinstance(dp, bool):
                    if dp:
                        deg_count(idx_ref, n)
                else:
                    @pl.when(dp)
                    def _():
                        deg_count(idx_ref, n)

            def step(i, s, s1, s2, first):
                if not first:
                    scatter_wait(s2)
                    idx_load(i + 2, s2)
                idx_wait(s1)
                gather_start(s1)
                gather_wait(s)
                scatter_start(s)
                maybe_deg(i, didx_b[s], CH)

            idx_load(0, 0)
            idx_load(1, 1)
            idx_wait(0)
            gather_start(0)
            idx_load(2, 2)
            step(0, 0, 1, 2, True)

            n_loop = n_main - 3
            peel = n_loop % 3
            for j in range(peel):
                step(1 + j, (1 + j) % 3, (2 + j) % 3, j % 3, False)
            i_base = 1 + peel

            def outer(o, _):
                i0 = i_base + o * 3
                for k in range(3):
                    sk = (i_base + k) % 3
                    step(i0 + k, sk, (sk + 1) % 3, (sk + 2) % 3, False)
                return 0

            lax.fori_loop(0, n_loop // 3, outer, 0, unroll=False)

            sa = (n_main - 2) % 3
            sb = (n_main - 1) % 3
            sc_ = (n_main - 3) % 3
            scatter_wait(sc_)
            idx_wait(sb)
            gather_start(sb)
            gather_wait(sa)
            scatter_start(sa)
            maybe_deg(n_main - 2, didx_b[sa], CH)
            gather_wait(sb)
            scatter_start(sb)
            maybe_deg(n_main - 1, didx_b[sb], CH)

            if tail:
                base = tbase + n_main * CH
                pltpu.sync_copy(srcv.at[pl.ds(base, tail)], sidx_t)
                pltpu.sync_copy(dstv.at[pl.ds(base, tail)], didx_t)
                pltpu.async_copy(feat.at[sidx_t], rows_b[sc_].at[pl.ds(0, tail)],
                                 sem).wait()
                pltpu.sync_copy(rows_b[sc_].at[pl.ds(0, tail)],
                                acc.at[didx_t], add=True)
                maybe_deg(n_main, didx_t, tail)
            scatter_wait(sa)
            scatter_wait(sb)

        @pl.when(cid == 0)
        def _():
            run_branch(feat1, lambda i: i < n_cut)

        @pl.when(cid == 1)
        def _():
            run_branch(feat2, lambda i: i >= n_cut)

        pltpu.sync_copy(degloc.at[pl.ds(0, N)],
                        degp.at[pl.ds((cid * NS + sid) * N, N)])

        plsc.subcore_barrier()

        n_rb = jnp.where(sid == NS - 1, last_rows // rb, rpt // rb)

        def rb_body(k, _):
            r0 = pl.multiple_of(base_r + k * rb, 8)
            pltpu.sync_copy(acc.at[pl.ds(r0, rb)], rows.at[pl.ds(0, rb)])

            @pl.when(cid == 0)
            def _():
                pltpu.sync_copy(rows.at[pl.ds(0, rb)], agg1.at[pl.ds(r0, rb)])

            @pl.when(cid == 1)
            def _():
                pltpu.sync_copy(rows.at[pl.ds(0, rb)], agg2.at[pl.ds(r0, rb)])

            return 0

        lax.fori_loop(0, n_rb, rb_body, 0, unroll=False)

    return sc_agg


def _metapath_acts(n_mp, agg1_ref, agg2_ref, deg_ref, wmp_ref):
    ones_p = jnp.ones((deg_ref.shape[1], 1), jnp.float32)
    deg = jnp.dot(deg_ref[:, :], ones_p,
                  preferred_element_type=jnp.float32)
    recip = 1.0 / jnp.maximum(deg, 1.0)
    a1 = agg1_ref[:, :] * recip
    a2 = agg2_ref[:, :] * recip
    h1s = []
    h2s = []

    def elu(x):
        return jnp.where(x > 0, x, jnp.exp(jnp.minimum(x, 0.0)) - 1.0)

    for m in range(n_mp):
        wm = wmp_ref[m, :, :]
        h1s.append(elu(jnp.dot(a1, wm, preferred_element_type=jnp.float32)))
        h2s.append(elu(jnp.dot(a2, wm, preferred_element_type=jnp.float32)))
    return h1s, h2s


def _tc_p0_body(N, n_mp,
                agg1_ref, agg2_ref, deg_ref, msk_ref, wmp_ref, wa_ref, bq_ref,
                md_ref,
                w1_ref, w2_ref, vv_ref,
                s1_ref, s2_ref, cs_ref, ms_ref):
    i = pl.program_id(0)
    nb = pl.num_programs(0)
    h1s, h2s = _metapath_acts(n_mp, agg1_ref, agg2_ref, deg_ref, wmp_ref)

    @pl.when(i == 0)
    def _():
        s1_ref[:, :] = jnp.zeros_like(s1_ref)
        s2_ref[:, :] = jnp.zeros_like(s2_ref)
        cs_ref[:, :] = jnp.zeros_like(cs_ref)
        ms_ref[:, :] = jnp.zeros_like(ms_ref)

    wa = wa_ref[:, :]
    b_attn = bq_ref[0:1, :]
    q_attn = bq_ref[1:2, :]
    mskb = msk_ref[:, :]
    for m in range(n_mp):
        t1 = jnp.tanh(jnp.dot(h1s[m], wa, preferred_element_type=jnp.float32)
                      + b_attn) * q_attn
        s1_ref[pl.ds(m, 1), :] += jnp.sum(t1, axis=0, keepdims=True)
        t2 = jnp.tanh(jnp.dot(h2s[m], wa, preferred_element_type=jnp.float32)
                      + b_attn) * q_attn
        s2_ref[pl.ds(m, 1), :] += jnp.sum(t2, axis=0, keepdims=True)
        cs_ref[pl.ds(m, 1), :] += jnp.sum(h1s[m] * mskb, axis=0,
                                          keepdims=True)
    ms_ref[:, :] += jnp.broadcast_to(
        jnp.sum(mskb).reshape(1, 1), ms_ref.shape)

    @pl.when(i == nb - 1)
    def _():
        for sacc, wref in ((s1_ref, w1_ref), (s2_ref, w2_ref)):
            sc = jnp.sum(sacc[:, :], axis=1, keepdims=True) / N
            mx = jnp.max(sc, axis=0, keepdims=True)
            e = jnp.exp(sc - mx)
            w = e / jnp.sum(e, axis=0, keepdims=True)
            wref[:, :] = jnp.broadcast_to(w, wref.shape)
        crow = (jnp.sum(w1_ref[:, :] * cs_ref[:, :], axis=0, keepdims=True)
                / ms_ref[0:1, :])
        cvec = 1.0 / (1.0 + jnp.exp(-crow))
        vv_ref[:, :] = lax.dot_general(
            cvec, md_ref[:, :], (((1,), (1,)), ((), ())),
            preferred_element_type=jnp.float32)


def _tc_p1_body(n_mp,
                agg1_ref, agg2_ref, deg_ref, wmp_ref, w1_ref, w2_ref, vv_ref,
                wfc_ref, bfc_ref,
                preds_ref, sc1_ref, sc2_ref):
    h1s, h2s = _metapath_acts(n_mp, agg1_ref, agg2_ref, deg_ref, wmp_ref)
    h1 = h1s[0] * w1_ref[pl.ds(0, 1), :]
    h2 = h2s[0] * w2_ref[pl.ds(0, 1), :]
    for m in range(1, n_mp):
        h1 = h1 + h1s[m] * w1_ref[pl.ds(m, 1), :]
        h2 = h2 + h2s[m] * w2_ref[pl.ds(m, 1), :]
    preds_ref[:, :] = (jnp.dot(h1, wfc_ref[:, :],
                               preferred_element_type=jnp.float32)
                       + bfc_ref[:, :])
    s1v = jnp.sum(h1 * vv_ref[:, :], axis=1, keepdims=True)
    s2v = jnp.sum(h2 * vv_ref[:, :], axis=1, keepdims=True)
    sc1_ref[:, :] = jnp.broadcast_to(s1v, sc1_ref.shape)
    sc2_ref[:, :] = jnp.broadcast_to(s2v, sc2_ref.shape)


def _build_tc_dense(N, D, n_mp, n_cls, B=1000):
    nb = N // B
    im_rows = lambda i: (i, 0)
    im_fixed = lambda i: (0, 0)

    p0 = pl.pallas_call(
        functools.partial(_tc_p0_body, N, n_mp),
        grid=(nb,),
        in_specs=[
            pl.BlockSpec((B, D), im_rows),
            pl.BlockSpec((B, D), im_rows),
            pl.BlockSpec((B, 32), im_rows),
            pl.BlockSpec((B, 1), im_rows),
            pl.BlockSpec((n_mp, D, D), lambda i: (0, 0, 0)),
            pl.BlockSpec((D, D), im_fixed),
            pl.BlockSpec((2, D), im_fixed),
            pl.BlockSpec((D, D), im_fixed),
        ],
        out_specs=[
            pl.BlockSpec((n_mp, D), im_fixed),
            pl.BlockSpec((n_mp, D), im_fixed),
            pl.BlockSpec((1, D), im_fixed),
        ],
        out_shape=[
            jax.ShapeDtypeStruct((n_mp, D), jnp.float32),
            jax.ShapeDtypeStruct((n_mp, D), jnp.float32),
            jax.ShapeDtypeStruct((1, D), jnp.float32),
        ],
        scratch_shapes=[
            pltpu.VMEM((n_mp, D), jnp.float32),
            pltpu.VMEM((n_mp, D), jnp.float32),
            pltpu.VMEM((n_mp, D), jnp.float32),
            pltpu.VMEM((1, D), jnp.float32),
        ],
        compiler_params=pltpu.CompilerParams(
            dimension_semantics=("arbitrary",)),
    )
    p1 = pl.pallas_call(
        functools.partial(_tc_p1_body, n_mp),
        grid=(nb,),
        in_specs=[
            pl.BlockSpec((B, D), im_rows),
            pl.BlockSpec((B, D), im_rows),
            pl.BlockSpec((B, 32), im_rows),
            pl.BlockSpec((n_mp, D, D), lambda i: (0, 0, 0)),
            pl.BlockSpec((n_mp, D), im_fixed),
            pl.BlockSpec((n_mp, D), im_fixed),
            pl.BlockSpec((1, D), im_fixed),
            pl.BlockSpec((D, n_cls), im_fixed),
            pl.BlockSpec((1, n_cls), im_fixed),
        ],
        out_specs=[
            pl.BlockSpec((B, n_cls), im_rows),
            pl.BlockSpec((B, 8), im_rows),
            pl.BlockSpec((B, 8), im_rows),
        ],
        out_shape=[
            jax.ShapeDtypeStruct((N, n_cls), jnp.float32),
            jax.ShapeDtypeStruct((N, 8), jnp.float32),
            jax.ShapeDtypeStruct((N, 8), jnp.float32),
        ],
        compiler_params=pltpu.CompilerParams(
            dimension_semantics=("arbitrary",)),
    )
    return p0, p1


def kernel(feat1, feat2, msk, samp_bias1, samp_bias2, edge_index,
           W_mp, W_attn, b_attn, q_attn, W_fc, b_fc, M_disc):
    N, D = feat1.shape
    E = edge_index.shape[1]
    n_mp = W_mp.shape[0]
    n_cls = W_fc.shape[1]

    src = edge_index[0]
    dst = edge_index[1]

    sc_agg = _build_sc_agg(N, D, E)
    agg1, agg2, degp = sc_agg(feat1, feat2, src, dst)
    degp2 = degp.reshape(32, N).T

    mskc = msk.reshape(N, 1)
    bq = jnp.stack([b_attn, q_attn])
    p0, p1 = _build_tc_dense(N, D, n_mp, n_cls)
    w3, w3b, vv = p0(agg1, agg2, degp2, mskc, W_mp, W_attn, bq, M_disc)
    preds, sc1f, sc2f = p1(agg1, agg2, degp2, W_mp, w3, w3b, vv,
                           W_fc, b_fc.reshape(1, n_cls))

    weights = w3[:, 0]
    sc_1 = sc1f[:, 0][None, :] + samp_bias1
    sc_2 = sc2f[:, 0][None, :] + samp_bias2
    reg = jnp.concatenate([sc_1, sc_2], axis=1)
    return (preds, weights, reg)


# speedup vs baseline: 9.4777x; 1.0110x over previous
import functools

import jax
import jax.numpy as jnp
from jax import lax
from jax.experimental import pallas as pl
from jax.experimental.pallas import tpu as pltpu
from jax.experimental.pallas import tpu_sc as plsc


def _build_sc_agg(N, D, E):
    NS = 16
    per_tile = E // NS
    CH = 80
    n_main, tail = divmod(per_tile, CH)
    n_cut = n_main // 2
    rpt = ((N + NS - 1) // NS + CH - 1) // CH * CH
    last_rows = N - (NS - 1) * rpt
    assert 0 < last_rows <= rpt and last_rows % 16 == 0
    nz_full = rpt // CH
    nz_last, z_tail = divmod(last_rows, CH)
    assert z_tail % 16 == 0
    rb = 80
    assert rpt % rb == 0 and last_rows % rb == 0 and rb <= CH

    mesh = plsc.VectorSubcoreMesh(core_axis_name="c", subcore_axis_name="s")

    @functools.partial(
        pl.kernel,
        out_type=[
            jax.ShapeDtypeStruct((N, D), jnp.float32),
            jax.ShapeDtypeStruct((N, D), jnp.float32),
            jax.ShapeDtypeStruct((2 * NS * N,), jnp.float32),
        ],
        mesh=mesh,
        scratch_types=[
            [pltpu.VMEM((CH,), jnp.int32)] * 3,
            [pltpu.VMEM((CH,), jnp.int32)] * 3,
            pltpu.VMEM((tail,), jnp.int32),
            pltpu.VMEM((tail,), jnp.int32),
            [pltpu.VMEM((CH, D), jnp.float32)] * 3,
            pltpu.VMEM((N + 16,), jnp.float32),
            pltpu.VMEM_SHARED((N, D), jnp.float32),
            [pltpu.SemaphoreType.DMA] * 3,
            [pltpu.SemaphoreType.DMA] * 3,
            [pltpu.SemaphoreType.DMA] * 3,
            [pltpu.SemaphoreType.DMA] * 3,
            pltpu.SemaphoreType.DMA,
        ],
    )
    def sc_agg(feat1, feat2, srcv, dstv, agg1, agg2, degp,
               sidx_b, didx_b, sidx_t, didx_t, rows_b, degloc, acc,
               sem_is, sem_id, sem_g, sem_s, sem):
        rows = rows_b[0]
        cid = lax.axis_index("c")
        sid = lax.axis_index("s")

        zero16 = jnp.zeros((16,), jnp.float32)
        one16 = jnp.ones((16,), jnp.float32)

        def fill_row(r, _):
            for g in range(D // 16):
                rows[r, pl.ds(g * 16, 16)] = zero16
            return 0

        lax.fori_loop(0, CH, fill_row, 0, unroll=False)

        def zero_deg(i, _):
            degloc[pl.ds(i * 16, 16)] = zero16
            return 0

        lax.fori_loop(0, (N + 16) // 16, zero_deg, 0, unroll=False)

        base_r = sid * rpt

        @pl.when(sid < NS - 1)
        def _():
            descs = [pltpu.async_copy(rows,
                                      acc.at[pl.ds(base_r + k * CH, CH)], sem)
                     for k in range(nz_full)]
            for d in descs:
                d.wait()

        @pl.when(sid == NS - 1)
        def _():
            b = (NS - 1) * rpt
            descs = [pltpu.async_copy(rows,
                                      acc.at[pl.ds(b + k * CH, CH)], sem)
                     for k in range(nz_last)]
            if z_tail:
                descs.append(pltpu.async_copy(
                    rows.at[pl.ds(0, z_tail)],
                    acc.at[pl.ds(b + nz_last * CH, z_tail)], sem))
            for d in descs:
                d.wait()

        plsc.subcore_barrier()

        tbase = sid * per_tile

        unit16 = jnp.where(lax.iota(jnp.int32, 16) == 0, 1.0, 0.0)

        def deg_count(idx_ref, n):
            for g in range(n // 16):
                idx16 = idx_ref[pl.ds(g * 16, 16)]
                for l in range(16):
                    d = idx16[l]
                    plsc.addupdate(degloc.at[pl.ds(d, 16)], unit16)

        def run_branch(feat, deg_pred):
            def idx_load(i, s):
                base = pl.multiple_of(tbase + i * CH, 8)
                pltpu.async_copy(srcv.at[pl.ds(base, CH)], sidx_b[s],
                                 sem_is[s])
                pltpu.async_copy(dstv.at[pl.ds(base, CH)], didx_b[s],
                                 sem_id[s])

            def idx_wait(s):
                pltpu.make_async_copy(srcv.at[pl.ds(0, CH)], sidx_b[s],
                                      sem_is[s]).wait()
                pltpu.make_async_copy(dstv.at[pl.ds(0, CH)], didx_b[s],
                                      sem_id[s]).wait()

            def gather_start(s):
                pltpu.async_copy(feat.at[sidx_b[s]], rows_b[s], sem_g[s])

            def gather_wait(s):
                pltpu.make_async_copy(feat.at[sidx_b[s]], rows_b[s],
                                      sem_g[s]).wait()

            def scatter_start(s):
                pltpu.async_copy(rows_b[s], acc.at[didx_b[s]], sem_s[s],
                                 add=True)

            def scatter_wait(s):
                pltpu.make_async_copy(rows_b[s], acc.at[didx_b[s]],
                                      sem_s[s]).wait()

            def maybe_deg(i, idx_ref, n):
                dp = deg_pred(i)
                if isinstance(dp, bool):
                    if dp:
                        deg_count(idx_ref, n)
                else:
                    @pl.when(dp)
                    def _():
                        deg_count(idx_ref, n)

            def step(i, s, s1, s2, first):
                if not first:
                    scatter_wait(s2)
                    idx_load(i + 2, s2)
                idx_wait(s1)
                gather_start(s1)
                gather_wait(s)
                scatter_start(s)
                maybe_deg(i, didx_b[s], CH)

            idx_load(0, 0)
            idx_load(1, 1)
            idx_wait(0)
            gather_start(0)
            idx_load(2, 2)
            step(0, 0, 1, 2, True)

            n_loop = n_main - 3
            peel = n_loop % 3
            for j in range(peel):
                step(1 + j, (1 + j) % 3, (2 + j) % 3, j % 3, False)
            i_base = 1 + peel

            def outer(o, _):
                i0 = i_base + o * 3
                for k in range(3):
                    sk = (i_base + k) % 3
                    step(i0 + k, sk, (sk + 1) % 3, (sk + 2) % 3, False)
                return 0

            lax.fori_loop(0, n_loop // 3, outer, 0, unroll=False)

            sa = (n_main - 2) % 3
            sb = (n_main - 1) % 3
            sc_ = (n_main - 3) % 3
            scatter_wait(sc_)
            idx_wait(sb)
            gather_start(sb)
            gather_wait(sa)
            scatter_start(sa)
            maybe_deg(n_main - 2, didx_b[sa], CH)
            gather_wait(sb)
            scatter_start(sb)
            maybe_deg(n_main - 1, didx_b[sb], CH)

            if tail:
                base = tbase + n_main * CH
                pltpu.sync_copy(srcv.at[pl.ds(base, tail)], sidx_t)
                pltpu.sync_copy(dstv.at[pl.ds(base, tail)], didx_t)
                pltpu.async_copy(feat.at[sidx_t], rows_b[sc_].at[pl.ds(0, tail)],
                                 sem).wait()
                pltpu.sync_copy(rows_b[sc_].at[pl.ds(0, tail)],
                                acc.at[didx_t], add=True)
                maybe_deg(n_main, didx_t, tail)
            scatter_wait(sa)
            scatter_wait(sb)

        @pl.when(cid == 0)
        def _():
            run_branch(feat1, lambda i: i < n_cut)

        @pl.when(cid == 1)
        def _():
            run_branch(feat2, lambda i: i >= n_cut)

        deg_pub = pltpu.async_copy(degloc.at[pl.ds(0, N)],
                                   degp.at[pl.ds((cid * NS + sid) * N, N)],
                                   sem)

        plsc.subcore_barrier()

        def writeback(agg_out, nk):
            def in_copy(k, s, start):
                r0 = base_r + k * rb
                args = (acc.at[pl.ds(r0, rb)], rows_b[s].at[pl.ds(0, rb)],
                        sem_g[s])
                return pltpu.async_copy(*args) if start else \
                    pltpu.make_async_copy(*args)

            def out_copy(k, s, start):
                r0 = base_r + k * rb
                args = (rows_b[s].at[pl.ds(0, rb)], agg_out.at[pl.ds(r0, rb)],
                        sem_s[s])
                return pltpu.async_copy(*args) if start else \
                    pltpu.make_async_copy(*args)

            for k in range(nk):
                s = k % 2
                if k >= 2:
                    out_copy(k - 2, s, False).wait()
                in_copy(k, s, True)
                in_copy(k, s, False).wait()
                out_copy(k, s, True)
            for k in (nk - 2, nk - 1):
                if 0 <= k:
                    out_copy(k, k % 2, False).wait()

        nk_full = rpt // rb
        nk_last = last_rows // rb
        for cc, agg_out in ((0, agg1), (1, agg2)):
            @pl.when(jnp.logical_and(cid == cc, sid < NS - 1))
            def _():
                writeback(agg_out, nk_full)

            @pl.when(jnp.logical_and(cid == cc, sid == NS - 1))
            def _():
                writeback(agg_out, nk_last)

        deg_pub.wait()

    return sc_agg


def _metapath_acts(n_mp, agg1_ref, agg2_ref, deg_ref, wmp_ref):
    ones_p = jnp.ones((deg_ref.shape[1], 1), jnp.float32)
    deg = jnp.dot(deg_ref[:, :], ones_p,
                  preferred_element_type=jnp.float32)
    recip = 1.0 / jnp.maximum(deg, 1.0)
    a1 = agg1_ref[:, :] * recip
    a2 = agg2_ref[:, :] * recip
    h1s = []
    h2s = []

    def elu(x):
        return jnp.where(x > 0, x, jnp.exp(jnp.minimum(x, 0.0)) - 1.0)

    for m in range(n_mp):
        wm = wmp_ref[m, :, :]
        h1s.append(elu(jnp.dot(a1, wm, preferred_element_type=jnp.float32)))
        h2s.append(elu(jnp.dot(a2, wm, preferred_element_type=jnp.float32)))
    return h1s, h2s


def _tc_p0_body(N, n_mp,
                agg1_ref, agg2_ref, deg_ref, msk_ref, wmp_ref, wa_ref, bq_ref,
                md_ref,
                w1_ref, w2_ref, vv_ref,
                s1_ref, s2_ref, cs_ref, ms_ref):
    i = pl.program_id(0)
    nb = pl.num_programs(0)
    h1s, h2s = _metapath_acts(n_mp, agg1_ref, agg2_ref, deg_ref, wmp_ref)

    @pl.when(i == 0)
    def _():
        s1_ref[:, :] = jnp.zeros_like(s1_ref)
        s2_ref[:, :] = jnp.zeros_like(s2_ref)
        cs_ref[:, :] = jnp.zeros_like(cs_ref)
        ms_ref[:, :] = jnp.zeros_like(ms_ref)

    wa = wa_ref[:, :]
    b_attn = bq_ref[0:1, :]
    q_attn = bq_ref[1:2, :]
    mskb = msk_ref[:, :]
    for m in range(n_mp):
        t1 = jnp.tanh(jnp.dot(h1s[m], wa, preferred_element_type=jnp.float32)
                      + b_attn) * q_attn
        s1_ref[pl.ds(m, 1), :] += jnp.sum(t1, axis=0, keepdims=True)
        t2 = jnp.tanh(jnp.dot(h2s[m], wa, preferred_element_type=jnp.float32)
                      + b_attn) * q_attn
        s2_ref[pl.ds(m, 1), :] += jnp.sum(t2, axis=0, keepdims=True)
        cs_ref[pl.ds(m, 1), :] += jnp.sum(h1s[m] * mskb, axis=0,
                                          keepdims=True)
    ms_ref[:, :] += jnp.broadcast_to(
        jnp.sum(mskb).reshape(1, 1), ms_ref.shape)

    @pl.when(i == nb - 1)
    def _():
        for sacc, wref in ((s1_ref, w1_ref), (s2_ref, w2_ref)):
            sc = jnp.sum(sacc[:, :], axis=1, keepdims=True) / N
            mx = jnp.max(sc, axis=0, keepdims=True)
            e = jnp.exp(sc - mx)
            w = e / jnp.sum(e, axis=0, keepdims=True)
            wref[:, :] = jnp.broadcast_to(w, wref.shape)
        crow = (jnp.sum(w1_ref[:, :] * cs_ref[:, :], axis=0, keepdims=True)
                / ms_ref[0:1, :])
        cvec = 1.0 / (1.0 + jnp.exp(-crow))
        vv_ref[:, :] = lax.dot_general(
            cvec, md_ref[:, :], (((1,), (1,)), ((), ())),
            preferred_element_type=jnp.float32)


def _tc_p1_body(n_mp,
                agg1_ref, agg2_ref, deg_ref, wmp_ref, w1_ref, w2_ref, vv_ref,
                wfc_ref, bfc_ref,
                preds_ref, sc1_ref, sc2_ref):
    h1s, h2s = _metapath_acts(n_mp, agg1_ref, agg2_ref, deg_ref, wmp_ref)
    h1 = h1s[0] * w1_ref[pl.ds(0, 1), :]
    h2 = h2s[0] * w2_ref[pl.ds(0, 1), :]
    for m in range(1, n_mp):
        h1 = h1 + h1s[m] * w1_ref[pl.ds(m, 1), :]
        h2 = h2 + h2s[m] * w2_ref[pl.ds(m, 1), :]
    preds_ref[:, :] = (jnp.dot(h1, wfc_ref[:, :],
                               preferred_element_type=jnp.float32)
                       + bfc_ref[:, :])
    s1v = jnp.sum(h1 * vv_ref[:, :], axis=1, keepdims=True)
    s2v = jnp.sum(h2 * vv_ref[:, :], axis=1, keepdims=True)
    sc1_ref[:, :] = jnp.broadcast_to(s1v, sc1_ref.shape)
    sc2_ref[:, :] = jnp.broadcast_to(s2v, sc2_ref.shape)


def _build_tc_dense(N, D, n_mp, n_cls, B=1000):
    nb = N // B
    im_rows = lambda i: (i, 0)
    im_fixed = lambda i: (0, 0)

    p0 = pl.pallas_call(
        functools.partial(_tc_p0_body, N, n_mp),
        grid=(nb,),
        in_specs=[
            pl.BlockSpec((B, D), im_rows),
            pl.BlockSpec((B, D), im_rows),
            pl.BlockSpec((B, 32), im_rows),
            pl.BlockSpec((B, 1), im_rows),
            pl.BlockSpec((n_mp, D, D), lambda i: (0, 0, 0)),
            pl.BlockSpec((D, D), im_fixed),
            pl.BlockSpec((2, D), im_fixed),
            pl.BlockSpec((D, D), im_fixed),
        ],
        out_specs=[
            pl.BlockSpec((n_mp, D), im_fixed),
            pl.BlockSpec((n_mp, D), im_fixed),
            pl.BlockSpec((1, D), im_fixed),
        ],
        out_shape=[
            jax.ShapeDtypeStruct((n_mp, D), jnp.float32),
            jax.ShapeDtypeStruct((n_mp, D), jnp.float32),
            jax.ShapeDtypeStruct((1, D), jnp.float32),
        ],
        scratch_shapes=[
            pltpu.VMEM((n_mp, D), jnp.float32),
            pltpu.VMEM((n_mp, D), jnp.float32),
            pltpu.VMEM((n_mp, D), jnp.float32),
            pltpu.VMEM((1, D), jnp.float32),
        ],
        compiler_params=pltpu.CompilerParams(
            dimension_semantics=("arbitrary",)),
    )
    p1 = pl.pallas_call(
        functools.partial(_tc_p1_body, n_mp),
        grid=(nb,),
        in_specs=[
            pl.BlockSpec((B, D), im_rows),
            pl.BlockSpec((B, D), im_rows),
            pl.BlockSpec((B, 32), im_rows),
            pl.BlockSpec((n_mp, D, D), lambda i: (0, 0, 0)),
            pl.BlockSpec((n_mp, D), im_fixed),
            pl.BlockSpec((n_mp, D), im_fixed),
            pl.BlockSpec((1, D), im_fixed),
            pl.BlockSpec((D, n_cls), im_fixed),
            pl.BlockSpec((1, n_cls), im_fixed),
        ],
        out_specs=[
            pl.BlockSpec((B, n_cls), im_rows),
            pl.BlockSpec((B, 8), im_rows),
            pl.BlockSpec((B, 8), im_rows),
        ],
        out_shape=[
            jax.ShapeDtypeStruct((N, n_cls), jnp.float32),
            jax.ShapeDtypeStruct((N, 8), jnp.float32),
            jax.ShapeDtypeStruct((N, 8), jnp.float32),
        ],
        compiler_params=pltpu.CompilerParams(
            dimension_semantics=("arbitrary",)),
    )
    return p0, p1


def kernel(feat1, feat2, msk, samp_bias1, samp_bias2, edge_index,
           W_mp, W_attn, b_attn, q_attn, W_fc, b_fc, M_disc):
    N, D = feat1.shape
    E = edge_index.shape[1]
    n_mp = W_mp.shape[0]
    n_cls = W_fc.shape[1]

    src = edge_index[0]
    dst = edge_index[1]

    sc_agg = _build_sc_agg(N, D, E)
    agg1, agg2, degp = sc_agg(feat1, feat2, src, dst)
    degp2 = degp.reshape(32, N).T

    mskc = msk.reshape(N, 1)
    bq = jnp.stack([b_attn, q_attn])
    p0, p1 = _build_tc_dense(N, D, n_mp, n_cls)
    w3, w3b, vv = p0(agg1, agg2, degp2, mskc, W_mp, W_attn, bq, M_disc)
    preds, sc1f, sc2f = p1(agg1, agg2, degp2, W_mp, w3, w3b, vv,
                           W_fc, b_fc.reshape(1, n_cls))

    weights = w3[:, 0]
    sc_1 = sc1f[:, 0][None, :] + samp_bias1
    sc_2 = sc2f[:, 0][None, :] + samp_bias2
    reg = jnp.concatenate([sc_1, sc_2], axis=1)
    return (preds, weights, reg)


# speedup vs baseline: 10.2522x; 1.0817x over previous
import functools

import jax
import jax.numpy as jnp
from jax import lax
from jax.experimental import pallas as pl
from jax.experimental.pallas import tpu as pltpu
from jax.experimental.pallas import tpu_sc as plsc


def _build_sc_agg(N, D, E):
    NS = 16
    per_tile = E // NS
    CH = 48
    n_main, tail = divmod(per_tile, CH)
    n_cut = n_main // 2
    rpt = ((N + NS - 1) // NS + 127) // 128 * 128
    last_rows = N - (NS - 1) * rpt
    assert 0 < last_rows <= rpt and last_rows % 16 == 0
    rb = 40
    assert rpt % rb == 0 and last_rows % rb == 0 and rb <= CH
    nz_full = rpt // rb
    nz_last = last_rows // rb

    mesh = plsc.VectorSubcoreMesh(core_axis_name="c", subcore_axis_name="s")

    @functools.partial(
        pl.kernel,
        out_type=[
            jax.ShapeDtypeStruct((N, D), jnp.float32),
            jax.ShapeDtypeStruct((N, D), jnp.float32),
            jax.ShapeDtypeStruct((2 * NS * N,), jnp.float32),
        ],
        mesh=mesh,
        scratch_types=[
            [pltpu.VMEM((CH,), jnp.int32)] * 5,
            [pltpu.VMEM((CH,), jnp.int32)] * 5,
            pltpu.VMEM((max(tail, 16),), jnp.int32),
            pltpu.VMEM((max(tail, 16),), jnp.int32),
            [pltpu.VMEM((CH, D), jnp.float32)] * 5,
            pltpu.VMEM((N + 16,), jnp.float32),
            pltpu.VMEM_SHARED((N, D), jnp.float32),
            [pltpu.SemaphoreType.DMA] * 5,
            [pltpu.SemaphoreType.DMA] * 5,
            [pltpu.SemaphoreType.DMA] * 5,
            [pltpu.SemaphoreType.DMA] * 5,
            pltpu.SemaphoreType.DMA,
        ],
    )
    def sc_agg(feat1, feat2, srcv, dstv, agg1, agg2, degp,
               sidx_b, didx_b, sidx_t, didx_t, rows_b, degloc, acc,
               sem_is, sem_id, sem_g, sem_s, sem):
        rows = rows_b[0]
        cid = lax.axis_index("c")
        sid = lax.axis_index("s")

        zero16 = jnp.zeros((16,), jnp.float32)
        one16 = jnp.ones((16,), jnp.float32)

        def fill_row(r, _):
            for g in range(D // 16):
                rows[r, pl.ds(g * 16, 16)] = zero16
            return 0

        lax.fori_loop(0, CH, fill_row, 0, unroll=False)

        def zero_deg(i, _):
            degloc[pl.ds(i * 16, 16)] = zero16
            return 0

        lax.fori_loop(0, (N + 16) // 16, zero_deg, 0, unroll=False)

        base_r = sid * rpt

        def zero_acc(nz):
            descs = [pltpu.async_copy(rows.at[pl.ds(0, rb)],
                                      acc.at[pl.ds(base_r + k * rb, rb)], sem)
                     for k in range(nz)]
            for d in descs:
                d.wait()

        @pl.when(sid < NS - 1)
        def _():
            zero_acc(nz_full)

        @pl.when(sid == NS - 1)
        def _():
            zero_acc(nz_last)

        plsc.subcore_barrier()

        tbase = sid * per_tile

        unit16 = jnp.where(lax.iota(jnp.int32, 16) == 0, 1.0, 0.0)

        def deg_count(idx_ref, n):
            for g in range(n // 16):
                idx16 = idx_ref[pl.ds(g * 16, 16)]
                for l in range(16):
                    d = idx16[l]
                    plsc.addupdate(degloc.at[pl.ds(d, 16)], unit16)

        def run_branch(feat, deg_pred):
            def idx_load(i, s):
                base = pl.multiple_of(tbase + i * CH, 8)
                pltpu.async_copy(srcv.at[pl.ds(base, CH)], sidx_b[s],
                                 sem_is[s])
                pltpu.async_copy(dstv.at[pl.ds(base, CH)], didx_b[s],
                                 sem_id[s])

            def idx_wait(s):
                pltpu.make_async_copy(srcv.at[pl.ds(0, CH)], sidx_b[s],
                                      sem_is[s]).wait()
                pltpu.make_async_copy(dstv.at[pl.ds(0, CH)], didx_b[s],
                                      sem_id[s]).wait()

            def gather_start(s):
                pltpu.async_copy(feat.at[sidx_b[s]], rows_b[s], sem_g[s])

            def gather_wait(s):
                pltpu.make_async_copy(feat.at[sidx_b[s]], rows_b[s],
                                      sem_g[s]).wait()

            def scatter_start(s):
                pltpu.async_copy(rows_b[s], acc.at[didx_b[s]], sem_s[s],
                                 add=True)

            def scatter_wait(s):
                pltpu.make_async_copy(rows_b[s], acc.at[didx_b[s]],
                                      sem_s[s]).wait()

            def maybe_deg(i, idx_ref, n):
                dp = deg_pred(i)
                if isinstance(dp, bool):
                    if dp:
                        deg_count(idx_ref, n)
                else:
                    @pl.when(dp)
                    def _():
                        deg_count(idx_ref, n)

            def step(i, s, first):
                sF = (s + 3) % 5
                if not first:
                    scatter_wait(sF)
                idx_load(i + 3, sF)
                sg = (s + 2) % 5
                idx_wait(sg)
                gather_start(sg)
                gather_wait(s)
                scatter_start(s)
                maybe_deg(i, didx_b[s], CH)

            assert n_main >= 8
            idx_load(0, 0)
            idx_load(1, 1)
            idx_load(2, 2)
            idx_wait(0)
            gather_start(0)
            idx_wait(1)
            gather_start(1)
            step(0, 0, True)
            step(1, 1, True)

            n_loop = n_main - 5
            peel = n_loop % 5
            for j in range(peel):
                step(2 + j, (2 + j) % 5, False)
            i_base = 2 + peel

            def outer(o, _):
                i0 = i_base + o * 5
                for k in range(5):
                    step(i0 + k, (i_base + k) % 5, False)
                return 0

            lax.fori_loop(0, n_loop // 5, outer, 0, unroll=False)

            for j in (3, 2, 1):
                i = n_main - j
                s = i % 5
                if j == 3:
                    sg = (s + 2) % 5
                    idx_wait(sg)
                    gather_start(sg)
                gather_wait(s)
                scatter_start(s)
                maybe_deg(i, didx_b[s], CH)

            if tail:
                base = tbase + n_main * CH
                pltpu.sync_copy(srcv.at[pl.ds(base, tail)], sidx_t)
                pltpu.sync_copy(dstv.at[pl.ds(base, tail)], didx_t)
                pltpu.async_copy(feat.at[sidx_t],
                                 rows_b[(n_main - 5) % 5].at[pl.ds(0, tail)],
                                 sem).wait()
                pltpu.sync_copy(rows_b[(n_main - 5) % 5].at[pl.ds(0, tail)],
                                acc.at[didx_t], add=True)
                maybe_deg(n_main, didx_t, tail)
            for j in (5, 4, 3, 2, 1):
                scatter_wait((n_main - j) % 5)

        @pl.when(cid == 0)
        def _():
            run_branch(feat1, lambda i: i < n_cut)

        @pl.when(cid == 1)
        def _():
            run_branch(feat2, lambda i: i >= n_cut)

        deg_pub = pltpu.async_copy(degloc.at[pl.ds(0, N)],
                                   degp.at[pl.ds((cid * NS + sid) * N, N)],
                                   sem)

        plsc.subcore_barrier()

        def writeback(agg_out, nk):
            def in_copy(k, s, start):
                r0 = base_r + k * rb
                args = (acc.at[pl.ds(r0, rb)], rows_b[s].at[pl.ds(0, rb)],
                        sem_g[s])
                return pltpu.async_copy(*args) if start else \
                    pltpu.make_async_copy(*args)

            def out_copy(k, s, start):
                r0 = base_r + k * rb
                args = (rows_b[s].at[pl.ds(0, rb)], agg_out.at[pl.ds(r0, rb)],
                        sem_s[s])
                return pltpu.async_copy(*args) if start else \
                    pltpu.make_async_copy(*args)

            for k in range(nk):
                s = k % 2
                if k >= 2:
                    out_copy(k - 2, s, False).wait()
                in_copy(k, s, True)
                in_copy(k, s, False).wait()
                out_copy(k, s, True)
            for k in (nk - 2, nk - 1):
                if 0 <= k:
                    out_copy(k, k % 2, False).wait()

        nk_full = rpt // rb
        nk_last = last_rows // rb
        for cc, agg_out in ((0, agg1), (1, agg2)):
            @pl.when(jnp.logical_and(cid == cc, sid < NS - 1))
            def _():
                writeback(agg_out, nk_full)

            @pl.when(jnp.logical_and(cid == cc, sid == NS - 1))
            def _():
                writeback(agg_out, nk_last)

        deg_pub.wait()

    return sc_agg


def _metapath_acts(n_mp, agg1_ref, agg2_ref, deg_ref, wmp_ref):
    ones_p = jnp.ones((deg_ref.shape[1], 1), jnp.float32)
    deg = jnp.dot(deg_ref[:, :], ones_p,
                  preferred_element_type=jnp.float32)
    recip = 1.0 / jnp.maximum(deg, 1.0)
    a1 = agg1_ref[:, :] * recip
    a2 = agg2_ref[:, :] * recip
    h1s = []
    h2s = []

    def elu(x):
        return jnp.where(x > 0, x, jnp.exp(jnp.minimum(x, 0.0)) - 1.0)

    for m in range(n_mp):
        wm = wmp_ref[m, :, :]
        h1s.append(elu(jnp.dot(a1, wm, preferred_element_type=jnp.float32)))
        h2s.append(elu(jnp.dot(a2, wm, preferred_element_type=jnp.float32)))
    return h1s, h2s


def _tc_p0_body(N, n_mp,
                agg1_ref, agg2_ref, deg_ref, msk_ref, wmp_ref, wa_ref, bq_ref,
                md_ref,
                w1_ref, w2_ref, vv_ref,
                s1_ref, s2_ref, cs_ref, ms_ref):
    i = pl.program_id(0)
    nb = pl.num_programs(0)
    h1s, h2s = _metapath_acts(n_mp, agg1_ref, agg2_ref, deg_ref, wmp_ref)

    @pl.when(i == 0)
    def _():
        s1_ref[:, :] = jnp.zeros_like(s1_ref)
        s2_ref[:, :] = jnp.zeros_like(s2_ref)
        cs_ref[:, :] = jnp.zeros_like(cs_ref)
        ms_ref[:, :] = jnp.zeros_like(ms_ref)

    wa = wa_ref[:, :]
    b_attn = bq_ref[0:1, :]
    q_attn = bq_ref[1:2, :]
    mskb = msk_ref[:, :]
    for m in range(n_mp):
        t1 = jnp.tanh(jnp.dot(h1s[m], wa, preferred_element_type=jnp.float32)
                      + b_attn) * q_attn
        s1_ref[pl.ds(m, 1), :] += jnp.sum(t1, axis=0, keepdims=True)
        t2 = jnp.tanh(jnp.dot(h2s[m], wa, preferred_element_type=jnp.float32)
                      + b_attn) * q_attn
        s2_ref[pl.ds(m, 1), :] += jnp.sum(t2, axis=0, keepdims=True)
        cs_ref[pl.ds(m, 1), :] += jnp.sum(h1s[m] * mskb, axis=0,
                                          keepdims=True)
    ms_ref[:, :] += jnp.broadcast_to(
        jnp.sum(mskb).reshape(1, 1), ms_ref.shape)

    @pl.when(i == nb - 1)
    def _():
        for sacc, wref in ((s1_ref, w1_ref), (s2_ref, w2_ref)):
            sc = jnp.sum(sacc[:, :], axis=1, keepdims=True) / N
            mx = jnp.max(sc, axis=0, keepdims=True)
            e = jnp.exp(sc - mx)
            w = e / jnp.sum(e, axis=0, keepdims=True)
            wref[:, :] = jnp.broadcast_to(w, wref.shape)
        crow = (jnp.sum(w1_ref[:, :] * cs_ref[:, :], axis=0, keepdims=True)
                / ms_ref[0:1, :])
        cvec = 1.0 / (1.0 + jnp.exp(-crow))
        vv_ref[:, :] = lax.dot_general(
            cvec, md_ref[:, :], (((1,), (1,)), ((), ())),
            preferred_element_type=jnp.float32)


def _tc_p1_body(n_mp,
                agg1_ref, agg2_ref, deg_ref, wmp_ref, w1_ref, w2_ref, vv_ref,
                wfc_ref, bfc_ref,
                preds_ref, sc1_ref, sc2_ref):
    h1s, h2s = _metapath_acts(n_mp, agg1_ref, agg2_ref, deg_ref, wmp_ref)
    h1 = h1s[0] * w1_ref[pl.ds(0, 1), :]
    h2 = h2s[0] * w2_ref[pl.ds(0, 1), :]
    for m in range(1, n_mp):
        h1 = h1 + h1s[m] * w1_ref[pl.ds(m, 1), :]
        h2 = h2 + h2s[m] * w2_ref[pl.ds(m, 1), :]
    preds_ref[:, :] = (jnp.dot(h1, wfc_ref[:, :],
                               preferred_element_type=jnp.float32)
                       + bfc_ref[:, :])
    s1v = jnp.sum(h1 * vv_ref[:, :], axis=1, keepdims=True)
    s2v = jnp.sum(h2 * vv_ref[:, :], axis=1, keepdims=True)
    sc1_ref[:, :] = jnp.broadcast_to(s1v, sc1_ref.shape)
    sc2_ref[:, :] = jnp.broadcast_to(s2v, sc2_ref.shape)


def _build_tc_dense(N, D, n_mp, n_cls, B=1000):
    nb = N // B
    im_rows = lambda i: (i, 0)
    im_fixed = lambda i: (0, 0)

    p0 = pl.pallas_call(
        functools.partial(_tc_p0_body, N, n_mp),
        grid=(nb,),
        in_specs=[
            pl.BlockSpec((B, D), im_rows),
            pl.BlockSpec((B, D), im_rows),
            pl.BlockSpec((B, 32), im_rows),
            pl.BlockSpec((B, 1), im_rows),
            pl.BlockSpec((n_mp, D, D), lambda i: (0, 0, 0)),
            pl.BlockSpec((D, D), im_fixed),
            pl.BlockSpec((2, D), im_fixed),
            pl.BlockSpec((D, D), im_fixed),
        ],
        out_specs=[
            pl.BlockSpec((n_mp, D), im_fixed),
            pl.BlockSpec((n_mp, D), im_fixed),
            pl.BlockSpec((1, D), im_fixed),
        ],
        out_shape=[
            jax.ShapeDtypeStruct((n_mp, D), jnp.float32),
            jax.ShapeDtypeStruct((n_mp, D), jnp.float32),
            jax.ShapeDtypeStruct((1, D), jnp.float32),
        ],
        scratch_shapes=[
            pltpu.VMEM((n_mp, D), jnp.float32),
            pltpu.VMEM((n_mp, D), jnp.float32),
            pltpu.VMEM((n_mp, D), jnp.float32),
            pltpu.VMEM((1, D), jnp.float32),
        ],
        compiler_params=pltpu.CompilerParams(
            dimension_semantics=("arbitrary",)),
    )
    p1 = pl.pallas_call(
        functools.partial(_tc_p1_body, n_mp),
        grid=(nb,),
        in_specs=[
            pl.BlockSpec((B, D), im_rows),
            pl.BlockSpec((B, D), im_rows),
            pl.BlockSpec((B, 32), im_rows),
            pl.BlockSpec((n_mp, D, D), lambda i: (0, 0, 0)),
            pl.BlockSpec((n_mp, D), im_fixed),
            pl.BlockSpec((n_mp, D), im_fixed),
            pl.BlockSpec((1, D), im_fixed),
            pl.BlockSpec((D, n_cls), im_fixed),
            pl.BlockSpec((1, n_cls), im_fixed),
        ],
        out_specs=[
            pl.BlockSpec((B, n_cls), im_rows),
            pl.BlockSpec((B, 8), im_rows),
            pl.BlockSpec((B, 8), im_rows),
        ],
        out_shape=[
            jax.ShapeDtypeStruct((N, n_cls), jnp.float32),
            jax.ShapeDtypeStruct((N, 8), jnp.float32),
            jax.ShapeDtypeStruct((N, 8), jnp.float32),
        ],
        compiler_params=pltpu.CompilerParams(
            dimension_semantics=("arbitrary",)),
    )
    return p0, p1


def kernel(feat1, feat2, msk, samp_bias1, samp_bias2, edge_index,
           W_mp, W_attn, b_attn, q_attn, W_fc, b_fc, M_disc):
    N, D = feat1.shape
    E = edge_index.shape[1]
    n_mp = W_mp.shape[0]
    n_cls = W_fc.shape[1]

    src = edge_index[0]
    dst = edge_index[1]

    sc_agg = _build_sc_agg(N, D, E)
    agg1, agg2, degp = sc_agg(feat1, feat2, src, dst)
    degp2 = degp.reshape(32, N).T

    mskc = msk.reshape(N, 1)
    bq = jnp.stack([b_attn, q_attn])
    p0, p1 = _build_tc_dense(N, D, n_mp, n_cls)
    w3, w3b, vv = p0(agg1, agg2, degp2, mskc, W_mp, W_attn, bq, M_disc)
    preds, sc1f, sc2f = p1(agg1, agg2, degp2, W_mp, w3, w3b, vv,
                           W_fc, b_fc.reshape(1, n_cls))

    weights = w3[:, 0]
    sc_1 = sc1f[:, 0][None, :] + samp_bias1
    sc_2 = sc2f[:, 0][None, :] + samp_bias2
    reg = jnp.concatenate([sc_1, sc_2], axis=1)
    return (preds, weights, reg)


# speedup vs baseline: 10.5545x; 1.0295x over previous
import functools

import jax
import jax.numpy as jnp
from jax import lax
from jax.experimental import pallas as pl
from jax.experimental.pallas import tpu as pltpu
from jax.experimental.pallas import tpu_sc as plsc


def _build_sc_agg(N, D, E):
    NS = 16
    per_tile = E // NS
    CH = 40
    NSLOT = 6
    IDXD = 4
    GD = 3
    n_main, tail = divmod(per_tile, CH)
    n_cut = n_main // 2
    rpt = ((N + NS - 1) // NS + 127) // 128 * 128
    last_rows = N - (NS - 1) * rpt
    assert 0 < last_rows <= rpt and last_rows % 16 == 0
    rb = 40
    assert rpt % rb == 0 and last_rows % rb == 0 and rb <= CH
    nz_full = rpt // rb
    nz_last = last_rows // rb

    mesh = plsc.VectorSubcoreMesh(core_axis_name="c", subcore_axis_name="s")

    @functools.partial(
        pl.kernel,
        out_type=[
            jax.ShapeDtypeStruct((N, D), jnp.float32),
            jax.ShapeDtypeStruct((N, D), jnp.float32),
            jax.ShapeDtypeStruct((2 * NS * N,), jnp.float32),
        ],
        mesh=mesh,
        scratch_types=[
            [pltpu.VMEM((CH,), jnp.int32)] * NSLOT,
            [pltpu.VMEM((CH,), jnp.int32)] * NSLOT,
            pltpu.VMEM((max(tail, 16),), jnp.int32),
            pltpu.VMEM((max(tail, 16),), jnp.int32),
            [pltpu.VMEM((CH, D), jnp.float32)] * NSLOT,
            pltpu.VMEM((N + 16,), jnp.float32),
            pltpu.VMEM_SHARED((N, D), jnp.float32),
            [pltpu.SemaphoreType.DMA] * NSLOT,
            [pltpu.SemaphoreType.DMA] * NSLOT,
            [pltpu.SemaphoreType.DMA] * NSLOT,
            [pltpu.SemaphoreType.DMA] * NSLOT,
            pltpu.SemaphoreType.DMA,
        ],
    )
    def sc_agg(feat1, feat2, srcv, dstv, agg1, agg2, degp,
               sidx_b, didx_b, sidx_t, didx_t, rows_b, degloc, acc,
               sem_is, sem_id, sem_g, sem_s, sem):
        rows = rows_b[0]
        cid = lax.axis_index("c")
        sid = lax.axis_index("s")

        zero16 = jnp.zeros((16,), jnp.float32)
        one16 = jnp.ones((16,), jnp.float32)

        def fill_row(r, _):
            for g in range(D // 16):
                rows[r, pl.ds(g * 16, 16)] = zero16
            return 0

        lax.fori_loop(0, CH, fill_row, 0, unroll=False)

        def zero_deg(i, _):
            degloc[pl.ds(i * 16, 16)] = zero16
            return 0

        lax.fori_loop(0, (N + 16) // 16, zero_deg, 0, unroll=False)

        base_r = sid * rpt

        def zero_acc(nz):
            descs = [pltpu.async_copy(rows.at[pl.ds(0, rb)],
                                      acc.at[pl.ds(base_r + k * rb, rb)], sem)
                     for k in range(nz)]
            for d in descs:
                d.wait()

        @pl.when(sid < NS - 1)
        def _():
            zero_acc(nz_full)

        @pl.when(sid == NS - 1)
        def _():
            zero_acc(nz_last)

        plsc.subcore_barrier()

        tbase = sid * per_tile

        unit16 = jnp.where(lax.iota(jnp.int32, 16) == 0, 1.0, 0.0)

        def deg_count(idx_ref, n):
            for g in range(n // 16):
                idx16 = idx_ref[pl.ds(g * 16, 16)]
                for l in range(16):
                    d = idx16[l]
                    plsc.addupdate(degloc.at[pl.ds(d, 16)], unit16)

        def run_branch(feat, deg_pred):
            def idx_load(i, s):
                base = pl.multiple_of(tbase + i * CH, 8)
                pltpu.async_copy(srcv.at[pl.ds(base, CH)], sidx_b[s],
                                 sem_is[s])
                pltpu.async_copy(dstv.at[pl.ds(base, CH)], didx_b[s],
                                 sem_id[s])

            def idx_wait(s):
                pltpu.make_async_copy(srcv.at[pl.ds(0, CH)], sidx_b[s],
                                      sem_is[s]).wait()
                pltpu.make_async_copy(dstv.at[pl.ds(0, CH)], didx_b[s],
                                      sem_id[s]).wait()

            def gather_start(s):
                pltpu.async_copy(feat.at[sidx_b[s]], rows_b[s], sem_g[s])

            def gather_wait(s):
                pltpu.make_async_copy(feat.at[sidx_b[s]], rows_b[s],
                                      sem_g[s]).wait()

            def scatter_start(s):
                pltpu.async_copy(rows_b[s], acc.at[didx_b[s]], sem_s[s],
                                 add=True)

            def scatter_wait(s):
                pltpu.make_async_copy(rows_b[s], acc.at[didx_b[s]],
                                      sem_s[s]).wait()

            def maybe_deg(i, idx_ref, n):
                dp = deg_pred(i)
                if isinstance(dp, bool):
                    if dp:
                        deg_count(idx_ref, n)
                else:
                    @pl.when(dp)
                    def _():
                        deg_count(idx_ref, n)

            def step(i, s, first):
                sF = (s + IDXD) % NSLOT
                if not first:
                    scatter_wait(sF)
                idx_load(i + IDXD, sF)
                sg = (s + GD) % NSLOT
                idx_wait(sg)
                gather_start(sg)
                gather_wait(s)
                scatter_start(s)
                maybe_deg(i, didx_b[s], CH)

            assert 2 <= GD < IDXD < NSLOT and n_main >= 2 * NSLOT
            for j in range(IDXD):
                idx_load(j, j)
            for j in range(GD):
                idx_wait(j)
                gather_start(j)
            for i in range(NSLOT - IDXD):
                step(i, i % NSLOT, True)

            i_base0 = NSLOT - IDXD
            n_loop = (n_main - IDXD) - i_base0
            peel = n_loop % NSLOT
            for j in range(peel):
                step(i_base0 + j, (i_base0 + j) % NSLOT, False)
            i_base = i_base0 + peel

            def outer(o, _):
                i0 = i_base + o * NSLOT
                for k in range(NSLOT):
                    step(i0 + k, (i_base + k) % NSLOT, False)
                return 0

            lax.fori_loop(0, n_loop // NSLOT, outer, 0, unroll=False)

            for j in range(IDXD, 0, -1):
                i = n_main - j
                s = i % NSLOT
                if i + GD <= n_main - 1:
                    sg = (s + GD) % NSLOT
                    idx_wait(sg)
                    gather_start(sg)
                gather_wait(s)
                scatter_start(s)
                maybe_deg(i, didx_b[s], CH)

            for j in range(NSLOT, 0, -1):
                scatter_wait((n_main - j) % NSLOT)
            if tail:
                base = tbase + n_main * CH
                pltpu.sync_copy(srcv.at[pl.ds(base, tail)], sidx_t)
                pltpu.sync_copy(dstv.at[pl.ds(base, tail)], didx_t)
                pltpu.async_copy(feat.at[sidx_t],
                                 rows_b[0].at[pl.ds(0, tail)],
                                 sem).wait()
                pltpu.sync_copy(rows_b[0].at[pl.ds(0, tail)],
                                acc.at[didx_t], add=True)
                maybe_deg(n_main, didx_t, tail)

        @pl.when(cid == 0)
        def _():
            run_branch(feat1, lambda i: i < n_cut)

        @pl.when(cid == 1)
        def _():
            run_branch(feat2, lambda i: i >= n_cut)

        deg_pub = pltpu.async_copy(degloc.at[pl.ds(0, N)],
                                   degp.at[pl.ds((cid * NS + sid) * N, N)],
                                   sem)

        plsc.subcore_barrier()

        def writeback(agg_out, nk):
            def in_copy(k, s, start):
                r0 = base_r + k * rb
                args = (acc.at[pl.ds(r0, rb)], rows_b[s].at[pl.ds(0, rb)],
                        sem_g[s])
                return pltpu.async_copy(*args) if start else \
                    pltpu.make_async_copy(*args)

            def out_copy(k, s, start):
                r0 = base_r + k * rb
                args = (rows_b[s].at[pl.ds(0, rb)], agg_out.at[pl.ds(r0, rb)],
                        sem_s[s])
                return pltpu.async_copy(*args) if start else \
                    pltpu.make_async_copy(*args)

            for k in range(nk):
                s = k % 2
                if k >= 2:
                    out_copy(k - 2, s, False).wait()
                in_copy(k, s, True)
                in_copy(k, s, False).wait()
                out_copy(k, s, True)
            for k in (nk - 2, nk - 1):
                if 0 <= k:
                    out_copy(k, k % 2, False).wait()

        nk_full = rpt // rb
        nk_last = last_rows // rb
        for cc, agg_out in ((0, agg1), (1, agg2)):
            @pl.when(jnp.logical_and(cid == cc, sid < NS - 1))
            def _():
                writeback(agg_out, nk_full)

            @pl.when(jnp.logical_and(cid == cc, sid == NS - 1))
            def _():
                writeback(agg_out, nk_last)

        deg_pub.wait()

    return sc_agg


def _metapath_acts(n_mp, agg1_ref, agg2_ref, deg_ref, wmp_ref):
    ones_p = jnp.ones((deg_ref.shape[1], 1), jnp.float32)
    deg = jnp.dot(deg_ref[:, :], ones_p,
                  preferred_element_type=jnp.float32)
    recip = 1.0 / jnp.maximum(deg, 1.0)
    a1 = agg1_ref[:, :] * recip
    a2 = agg2_ref[:, :] * recip
    h1s = []
    h2s = []

    def elu(x):
        return jnp.where(x > 0, x, jnp.exp(jnp.minimum(x, 0.0)) - 1.0)

    for m in range(n_mp):
        wm = wmp_ref[m, :, :]
        h1s.append(elu(jnp.dot(a1, wm, preferred_element_type=jnp.float32)))
        h2s.append(elu(jnp.dot(a2, wm, preferred_element_type=jnp.float32)))
    return h1s, h2s


def _tc_p0_body(N, n_mp,
                agg1_ref, agg2_ref, deg_ref, msk_ref, wmp_ref, wa_ref, bq_ref,
                md_ref,
                w1_ref, w2_ref, vv_ref,
                s1_ref, s2_ref, cs_ref, ms_ref):
    i = pl.program_id(0)
    nb = pl.num_programs(0)
    h1s, h2s = _metapath_acts(n_mp, agg1_ref, agg2_ref, deg_ref, wmp_ref)

    @pl.when(i == 0)
    def _():
        s1_ref[:, :] = jnp.zeros_like(s1_ref)
        s2_ref[:, :] = jnp.zeros_like(s2_ref)
        cs_ref[:, :] = jnp.zeros_like(cs_ref)
        ms_ref[:, :] = jnp.zeros_like(ms_ref)

    wa = wa_ref[:, :]
    b_attn = bq_ref[0:1, :]
    q_attn = bq_ref[1:2, :]
    mskb = msk_ref[:, :]
    for m in range(n_mp):
        t1 = jnp.tanh(jnp.dot(h1s[m], wa, preferred_element_type=jnp.float32)
                      + b_attn) * q_attn
        s1_ref[pl.ds(m, 1), :] += jnp.sum(t1, axis=0, keepdims=True)
        t2 = jnp.tanh(jnp.dot(h2s[m], wa, preferred_element_type=jnp.float32)
                      + b_attn) * q_attn
        s2_ref[pl.ds(m, 1), :] += jnp.sum(t2, axis=0, keepdims=True)
        cs_ref[pl.ds(m, 1), :] += jnp.sum(h1s[m] * mskb, axis=0,
                                          keepdims=True)
    ms_ref[:, :] += jnp.broadcast_to(
        jnp.sum(mskb).reshape(1, 1), ms_ref.shape)

    @pl.when(i == nb - 1)
    def _():
        for sacc, wref in ((s1_ref, w1_ref), (s2_ref, w2_ref)):
            sc = jnp.sum(sacc[:, :], axis=1, keepdims=True) / N
            mx = jnp.max(sc, axis=0, keepdims=True)
            e = jnp.exp(sc - mx)
            w = e / jnp.sum(e, axis=0, keepdims=True)
            wref[:, :] = jnp.broadcast_to(w, wref.shape)
        crow = (jnp.sum(w1_ref[:, :] * cs_ref[:, :], axis=0, keepdims=True)
                / ms_ref[0:1, :])
        cvec = 1.0 / (1.0 + jnp.exp(-crow))
        vv_ref[:, :] = lax.dot_general(
            cvec, md_ref[:, :], (((1,), (1,)), ((), ())),
            preferred_element_type=jnp.float32)


def _tc_p1_body(n_mp,
                agg1_ref, agg2_ref, deg_ref, wmp_ref, w1_ref, w2_ref, vv_ref,
                wfc_ref, bfc_ref,
                preds_ref, sc1_ref, sc2_ref):
    h1s, h2s = _metapath_acts(n_mp, agg1_ref, agg2_ref, deg_ref, wmp_ref)
    h1 = h1s[0] * w1_ref[pl.ds(0, 1), :]
    h2 = h2s[0] * w2_ref[pl.ds(0, 1), :]
    for m in range(1, n_mp):
        h1 = h1 + h1s[m] * w1_ref[pl.ds(m, 1), :]
        h2 = h2 + h2s[m] * w2_ref[pl.ds(m, 1), :]
    preds_ref[:, :] = (jnp.dot(h1, wfc_ref[:, :],
                               preferred_element_type=jnp.float32)
                       + bfc_ref[:, :])
    s1v = jnp.sum(h1 * vv_ref[:, :], axis=1, keepdims=True)
    s2v = jnp.sum(h2 * vv_ref[:, :], axis=1, keepdims=True)
    sc1_ref[:, :] = jnp.broadcast_to(s1v, sc1_ref.shape)
    sc2_ref[:, :] = jnp.broadcast_to(s2v, sc2_ref.shape)


def _build_tc_dense(N, D, n_mp, n_cls, B=1000):
    nb = N // B
    im_rows = lambda i: (i, 0)
    im_fixed = lambda i: (0, 0)

    p0 = pl.pallas_call(
        functools.partial(_tc_p0_body, N, n_mp),
        grid=(nb,),
        in_specs=[
            pl.BlockSpec((B, D), im_rows),
            pl.BlockSpec((B, D), im_rows),
            pl.BlockSpec((B, 32), im_rows),
            pl.BlockSpec((B, 1), im_rows),
            pl.BlockSpec((n_mp, D, D), lambda i: (0, 0, 0)),
            pl.BlockSpec((D, D), im_fixed),
            pl.BlockSpec((2, D), im_fixed),
            pl.BlockSpec((D, D), im_fixed),
        ],
        out_specs=[
            pl.BlockSpec((n_mp, D), im_fixed),
            pl.BlockSpec((n_mp, D), im_fixed),
            pl.BlockSpec((1, D), im_fixed),
        ],
        out_shape=[
            jax.ShapeDtypeStruct((n_mp, D), jnp.float32),
            jax.ShapeDtypeStruct((n_mp, D), jnp.float32),
            jax.ShapeDtypeStruct((1, D), jnp.float32),
        ],
        scratch_shapes=[
            pltpu.VMEM((n_mp, D), jnp.float32),
            pltpu.VMEM((n_mp, D), jnp.float32),
            pltpu.VMEM((n_mp, D), jnp.float32),
            pltpu.VMEM((1, D), jnp.float32),
        ],
        compiler_params=pltpu.CompilerParams(
            dimension_semantics=("arbitrary",)),
    )
    p1 = pl.pallas_call(
        functools.partial(_tc_p1_body, n_mp),
        grid=(nb,),
        in_specs=[
            pl.BlockSpec((B, D), im_rows),
            pl.BlockSpec((B, D), im_rows),
            pl.BlockSpec((B, 32), im_rows),
            pl.BlockSpec((n_mp, D, D), lambda i: (0, 0, 0)),
            pl.BlockSpec((n_mp, D), im_fixed),
            pl.BlockSpec((n_mp, D), im_fixed),
            pl.BlockSpec((1, D), im_fixed),
            pl.BlockSpec((D, n_cls), im_fixed),
            pl.BlockSpec((1, n_cls), im_fixed),
        ],
        out_specs=[
            pl.BlockSpec((B, n_cls), im_rows),
            pl.BlockSpec((B, 8), im_rows),
            pl.BlockSpec((B, 8), im_rows),
        ],
        out_shape=[
            jax.ShapeDtypeStruct((N, n_cls), jnp.float32),
            jax.ShapeDtypeStruct((N, 8), jnp.float32),
            jax.ShapeDtypeStruct((N, 8), jnp.float32),
        ],
        compiler_params=pltpu.CompilerParams(
            dimension_semantics=("arbitrary",)),
    )
    return p0, p1


def kernel(feat1, feat2, msk, samp_bias1, samp_bias2, edge_index,
           W_mp, W_attn, b_attn, q_attn, W_fc, b_fc, M_disc):
    N, D = feat1.shape
    E = edge_index.shape[1]
    n_mp = W_mp.shape[0]
    n_cls = W_fc.shape[1]

    src = edge_index[0]
    dst = edge_index[1]

    sc_agg = _build_sc_agg(N, D, E)
    agg1, agg2, degp = sc_agg(feat1, feat2, src, dst)
    degp2 = degp.reshape(32, N).T

    mskc = msk.reshape(N, 1)
    bq = jnp.stack([b_attn, q_attn])
    p0, p1 = _build_tc_dense(N, D, n_mp, n_cls)
    w3, w3b, vv = p0(agg1, agg2, degp2, mskc, W_mp, W_attn, bq, M_disc)
    preds, sc1f, sc2f = p1(agg1, agg2, degp2, W_mp, w3, w3b, vv,
                           W_fc, b_fc.reshape(1, n_cls))

    weights = w3[:, 0]
    sc_1 = sc1f[:, 0][None, :] + samp_bias1
    sc_2 = sc2f[:, 0][None, :] + samp_bias2
    reg = jnp.concatenate([sc_1, sc_2], axis=1)
    return (preds, weights, reg)


# speedup vs baseline: 10.7464x; 1.0182x over previous
import functools

import jax
import jax.numpy as jnp
from jax import lax
from jax.experimental import pallas as pl
from jax.experimental.pallas import tpu as pltpu
from jax.experimental.pallas import tpu_sc as plsc


def _build_sc_agg(N, D, E):
    NS = 16
    per_tile = E // NS
    CH = 48
    NSLOT = 6
    IDXD = 4
    GD = 3
    assert CH % 16 == 0
    n_main, tail = divmod(per_tile, CH)
    assert tail % 16 == 0
    n_cut = n_main // 2
    rpt = ((N + NS - 1) // NS + 127) // 128 * 128
    last_rows = N - (NS - 1) * rpt
    assert 0 < last_rows <= rpt and last_rows % 16 == 0
    rb = 40
    assert rpt % rb == 0 and last_rows % rb == 0 and rb <= CH
    nz_full = rpt // rb
    nz_last = last_rows // rb

    mesh = plsc.VectorSubcoreMesh(core_axis_name="c", subcore_axis_name="s")

    @functools.partial(
        pl.kernel,
        out_type=[
            jax.ShapeDtypeStruct((N, D), jnp.float32),
            jax.ShapeDtypeStruct((N, D), jnp.float32),
            jax.ShapeDtypeStruct((2 * NS * N,), jnp.float32),
        ],
        mesh=mesh,
        scratch_types=[
            [pltpu.VMEM((CH,), jnp.int32)] * NSLOT,
            [pltpu.VMEM((CH,), jnp.int32)] * NSLOT,
            pltpu.VMEM((max(tail, 16),), jnp.int32),
            pltpu.VMEM((max(tail, 16),), jnp.int32),
            [pltpu.VMEM((CH, D), jnp.float32)] * NSLOT,
            pltpu.VMEM((N + 16,), jnp.float32),
            pltpu.VMEM_SHARED((N, D), jnp.float32),
            [pltpu.SemaphoreType.DMA] * NSLOT,
            [pltpu.SemaphoreType.DMA] * NSLOT,
            [pltpu.SemaphoreType.DMA] * NSLOT,
            [pltpu.SemaphoreType.DMA] * NSLOT,
            pltpu.SemaphoreType.DMA,
        ],
    )
    def sc_agg(feat1, feat2, srcv, dstv, agg1, agg2, degp,
               sidx_b, didx_b, sidx_t, didx_t, rows_b, degloc, acc,
               sem_is, sem_id, sem_g, sem_s, sem):
        rows = rows_b[0]
        cid = lax.axis_index("c")
        sid = lax.axis_index("s")

        zero16 = jnp.zeros((16,), jnp.float32)
        one16 = jnp.ones((16,), jnp.float32)

        def fill_row(r, _):
            for g in range(D // 16):
                rows[r, pl.ds(g * 16, 16)] = zero16
            return 0

        lax.fori_loop(0, CH, fill_row, 0, unroll=False)

        def zero_deg(i, _):
            degloc[pl.ds(i * 16, 16)] = zero16
            return 0

        lax.fori_loop(0, (N + 16) // 16, zero_deg, 0, unroll=False)

        base_r = sid * rpt

        def zero_acc(nz):
            descs = [pltpu.async_copy(rows.at[pl.ds(0, rb)],
                                      acc.at[pl.ds(base_r + k * rb, rb)], sem)
                     for k in range(nz)]
            for d in descs:
                d.wait()

        @pl.when(sid < NS - 1)
        def _():
            zero_acc(nz_full)

        @pl.when(sid == NS - 1)
        def _():
            zero_acc(nz_last)

        plsc.subcore_barrier()

        tbase = sid * per_tile

        unit16 = jnp.where(lax.iota(jnp.int32, 16) == 0, 1.0, 0.0)

        def deg_count(idx_ref, n):
            for g in range(n // 16):
                idx16 = idx_ref[pl.ds(g * 16, 16)]
                for l in range(16):
                    d = idx16[l]
                    plsc.addupdate(degloc.at[pl.ds(d, 16)], unit16)

        def run_branch(feat, deg_pred):
            def idx_load(i, s):
                base = pl.multiple_of(tbase + i * CH, 8)
                pltpu.async_copy(srcv.at[pl.ds(base, CH)], sidx_b[s],
                                 sem_is[s])
                pltpu.async_copy(dstv.at[pl.ds(base, CH)], didx_b[s],
                                 sem_id[s])

            def idx_wait(s):
                pltpu.make_async_copy(srcv.at[pl.ds(0, CH)], sidx_b[s],
                                      sem_is[s]).wait()
                pltpu.make_async_copy(dstv.at[pl.ds(0, CH)], didx_b[s],
                                      sem_id[s]).wait()

            def gather_start(s):
                pltpu.async_copy(feat.at[sidx_b[s]], rows_b[s], sem_g[s])

            def gather_wait(s):
                pltpu.make_async_copy(feat.at[sidx_b[s]], rows_b[s],
                                      sem_g[s]).wait()

            def scatter_start(s):
                pltpu.async_copy(rows_b[s], acc.at[didx_b[s]], sem_s[s],
                                 add=True)

            def scatter_wait(s):
                pltpu.make_async_copy(rows_b[s], acc.at[didx_b[s]],
                                      sem_s[s]).wait()

            def maybe_deg(i, idx_ref, n):
                dp = deg_pred(i)
                if isinstance(dp, bool):
                    if dp:
                        deg_count(idx_ref, n)
                else:
                    @pl.when(dp)
                    def _():
                        deg_count(idx_ref, n)

            def step(i, s, first):
                sF = (s + IDXD) % NSLOT
                if not first:
                    scatter_wait(sF)
                idx_load(i + IDXD, sF)
                sg = (s + GD) % NSLOT
                idx_wait(sg)
                gather_start(sg)
                gather_wait(s)
                scatter_start(s)
                maybe_deg(i, didx_b[s], CH)

            assert 2 <= GD < IDXD < NSLOT and n_main >= 2 * NSLOT
            for j in range(IDXD):
                idx_load(j, j)
            for j in range(GD):
                idx_wait(j)
                gather_start(j)
            for i in range(NSLOT - IDXD):
                step(i, i % NSLOT, True)

            i_base0 = NSLOT - IDXD
            n_loop = (n_main - IDXD) - i_base0
            peel = n_loop % NSLOT
            for j in range(peel):
                step(i_base0 + j, (i_base0 + j) % NSLOT, False)
            i_base = i_base0 + peel

            def outer(o, _):
                i0 = i_base + o * NSLOT
                for k in range(NSLOT):
                    step(i0 + k, (i_base + k) % NSLOT, False)
                return 0

            lax.fori_loop(0, n_loop // NSLOT, outer, 0, unroll=False)

            for j in range(IDXD, 0, -1):
                i = n_main - j
                s = i % NSLOT
                if i + GD <= n_main - 1:
                    sg = (s + GD) % NSLOT
                    idx_wait(sg)
                    gather_start(sg)
                gather_wait(s)
                scatter_start(s)
                maybe_deg(i, didx_b[s], CH)

            for j in range(NSLOT, 0, -1):
                scatter_wait((n_main - j) % NSLOT)
            if tail:
                base = tbase + n_main * CH
                pltpu.sync_copy(srcv.at[pl.ds(base, tail)], sidx_t)
                pltpu.sync_copy(dstv.at[pl.ds(base, tail)], didx_t)
                pltpu.async_copy(feat.at[sidx_t],
                                 rows_b[0].at[pl.ds(0, tail)],
                                 sem).wait()
                pltpu.sync_copy(rows_b[0].at[pl.ds(0, tail)],
                                acc.at[didx_t], add=True)
                maybe_deg(n_main, didx_t, tail)

        @pl.when(cid == 0)
        def _():
            run_branch(feat1, lambda i: i < n_cut)

        @pl.when(cid == 1)
        def _():
            run_branch(feat2, lambda i: i >= n_cut)

        deg_pub = pltpu.async_copy(degloc.at[pl.ds(0, N)],
                                   degp.at[pl.ds((cid * NS + sid) * N, N)],
                                   sem)

        plsc.subcore_barrier()

        def writeback(agg_out, nk):
            def in_copy(k, s, start):
                r0 = base_r + k * rb
                args = (acc.at[pl.ds(r0, rb)], rows_b[s].at[pl.ds(0, rb)],
                        sem_g[s])
                return pltpu.async_copy(*args) if start else \
                    pltpu.make_async_copy(*args)

            def out_copy(k, s, start):
                r0 = base_r + k * rb
                args = (rows_b[s].at[pl.ds(0, rb)], agg_out.at[pl.ds(r0, rb)],
                        sem_s[s])
                return pltpu.async_copy(*args) if start else \
                    pltpu.make_async_copy(*args)

            for k in range(nk):
                s = k % 2
                if k >= 2:
                    out_copy(k - 2, s, False).wait()
                in_copy(k, s, True)
                in_copy(k, s, False).wait()
                out_copy(k, s, True)
            for k in (nk - 2, nk - 1):
                if 0 <= k:
                    out_copy(k, k % 2, False).wait()

        nk_full = rpt // rb
        nk_last = last_rows // rb
        for cc, agg_out in ((0, agg1), (1, agg2)):
            @pl.when(jnp.logical_and(cid == cc, sid < NS - 1))
            def _():
                writeback(agg_out, nk_full)

            @pl.when(jnp.logical_and(cid == cc, sid == NS - 1))
            def _():
                writeback(agg_out, nk_last)

        deg_pub.wait()

    return sc_agg


def _metapath_acts(n_mp, agg1_ref, agg2_ref, deg_ref, wmp_ref):
    ones_p = jnp.ones((deg_ref.shape[1], 1), jnp.float32)
    deg = jnp.dot(deg_ref[:, :], ones_p,
                  preferred_element_type=jnp.float32)
    recip = 1.0 / jnp.maximum(deg, 1.0)
    a1 = agg1_ref[:, :] * recip
    a2 = agg2_ref[:, :] * recip
    h1s = []
    h2s = []

    def elu(x):
        return jnp.where(x > 0, x, jnp.exp(jnp.minimum(x, 0.0)) - 1.0)

    for m in range(n_mp):
        wm = wmp_ref[m, :, :]
        h1s.append(elu(jnp.dot(a1, wm, preferred_element_type=jnp.float32)))
        h2s.append(elu(jnp.dot(a2, wm, preferred_element_type=jnp.float32)))
    return h1s, h2s


def _tc_p0_body(N, n_mp,
                agg1_ref, agg2_ref, deg_ref, msk_ref, wmp_ref, wa_ref, bq_ref,
                md_ref,
                w1_ref, w2_ref, vv_ref,
                s1_ref, s2_ref, cs_ref, ms_ref):
    i = pl.program_id(0)
    nb = pl.num_programs(0)
    h1s, h2s = _metapath_acts(n_mp, agg1_ref, agg2_ref, deg_ref, wmp_ref)

    @pl.when(i == 0)
    def _():
        s1_ref[:, :] = jnp.zeros_like(s1_ref)
        s2_ref[:, :] = jnp.zeros_like(s2_ref)
        cs_ref[:, :] = jnp.zeros_like(cs_ref)
        ms_ref[:, :] = jnp.zeros_like(ms_ref)

    wa = wa_ref[:, :]
    b_attn = bq_ref[0:1, :]
    q_attn = bq_ref[1:2, :]
    mskb = msk_ref[:, :]
    for m in range(n_mp):
        t1 = jnp.tanh(jnp.dot(h1s[m], wa, preferred_element_type=jnp.float32)
                      + b_attn) * q_attn
        s1_ref[pl.ds(m, 1), :] += jnp.sum(t1, axis=0, keepdims=True)
        t2 = jnp.tanh(jnp.dot(h2s[m], wa, preferred_element_type=jnp.float32)
                      + b_attn) * q_attn
        s2_ref[pl.ds(m, 1), :] += jnp.sum(t2, axis=0, keepdims=True)
        cs_ref[pl.ds(m, 1), :] += jnp.sum(h1s[m] * mskb, axis=0,
                                          keepdims=True)
    ms_ref[:, :] += jnp.broadcast_to(
        jnp.sum(mskb).reshape(1, 1), ms_ref.shape)

    @pl.when(i == nb - 1)
    def _():
        for sacc, wref in ((s1_ref, w1_ref), (s2_ref, w2_ref)):
            sc = jnp.sum(sacc[:, :], axis=1, keepdims=True) / N
            mx = jnp.max(sc, axis=0, keepdims=True)
            e = jnp.exp(sc - mx)
            w = e / jnp.sum(e, axis=0, keepdims=True)
            wref[:, :] = jnp.broadcast_to(w, wref.shape)
        crow = (jnp.sum(w1_ref[:, :] * cs_ref[:, :], axis=0, keepdims=True)
                / ms_ref[0:1, :])
        cvec = 1.0 / (1.0 + jnp.exp(-crow))
        vv_ref[:, :] = lax.dot_general(
            cvec, md_ref[:, :], (((1,), (1,)), ((), ())),
            preferred_element_type=jnp.float32)


def _tc_p1_body(n_mp,
                agg1_ref, agg2_ref, deg_ref, wmp_ref, w1_ref, w2_ref, vv_ref,
                wfc_ref, bfc_ref,
                preds_ref, sc1_ref, sc2_ref):
    h1s, h2s = _metapath_acts(n_mp, agg1_ref, agg2_ref, deg_ref, wmp_ref)
    h1 = h1s[0] * w1_ref[pl.ds(0, 1), :]
    h2 = h2s[0] * w2_ref[pl.ds(0, 1), :]
    for m in range(1, n_mp):
        h1 = h1 + h1s[m] * w1_ref[pl.ds(m, 1), :]
        h2 = h2 + h2s[m] * w2_ref[pl.ds(m, 1), :]
    preds_ref[:, :] = (jnp.dot(h1, wfc_ref[:, :],
                               preferred_element_type=jnp.float32)
                       + bfc_ref[:, :])
    s1v = jnp.sum(h1 * vv_ref[:, :], axis=1, keepdims=True)
    s2v = jnp.sum(h2 * vv_ref[:, :], axis=1, keepdims=True)
    sc1_ref[:, :] = jnp.broadcast_to(s1v, sc1_ref.shape)
    sc2_ref[:, :] = jnp.broadcast_to(s2v, sc2_ref.shape)


def _build_tc_dense(N, D, n_mp, n_cls, B=1000):
    nb = N // B
    im_rows = lambda i: (i, 0)
    im_fixed = lambda i: (0, 0)

    p0 = pl.pallas_call(
        functools.partial(_tc_p0_body, N, n_mp),
        grid=(nb,),
        in_specs=[
            pl.BlockSpec((B, D), im_rows),
            pl.BlockSpec((B, D), im_rows),
            pl.BlockSpec((B, 32), im_rows),
            pl.BlockSpec((B, 1), im_rows),
            pl.BlockSpec((n_mp, D, D), lambda i: (0, 0, 0)),
            pl.BlockSpec((D, D), im_fixed),
            pl.BlockSpec((2, D), im_fixed),
            pl.BlockSpec((D, D), im_fixed),
        ],
        out_specs=[
            pl.BlockSpec((n_mp, D), im_fixed),
            pl.BlockSpec((n_mp, D), im_fixed),
            pl.BlockSpec((1, D), im_fixed),
        ],
        out_shape=[
            jax.ShapeDtypeStruct((n_mp, D), jnp.float32),
            jax.ShapeDtypeStruct((n_mp, D), jnp.float32),
            jax.ShapeDtypeStruct((1, D), jnp.float32),
        ],
        scratch_shapes=[
            pltpu.VMEM((n_mp, D), jnp.float32),
            pltpu.VMEM((n_mp, D), jnp.float32),
            pltpu.VMEM((n_mp, D), jnp.float32),
            pltpu.VMEM((1, D), jnp.float32),
        ],
        compiler_params=pltpu.CompilerParams(
            dimension_semantics=("arbitrary",)),
    )
    p1 = pl.pallas_call(
        functools.partial(_tc_p1_body, n_mp),
        grid=(nb,),
        in_specs=[
            pl.BlockSpec((B, D), im_rows),
            pl.BlockSpec((B, D), im_rows),
            pl.BlockSpec((B, 32), im_rows),
            pl.BlockSpec((n_mp, D, D), lambda i: (0, 0, 0)),
            pl.BlockSpec((n_mp, D), im_fixed),
            pl.BlockSpec((n_mp, D), im_fixed),
            pl.BlockSpec((1, D), im_fixed),
            pl.BlockSpec((D, n_cls), im_fixed),
            pl.BlockSpec((1, n_cls), im_fixed),
        ],
        out_specs=[
            pl.BlockSpec((B, n_cls), im_rows),
            pl.BlockSpec((B, 8), im_rows),
            pl.BlockSpec((B, 8), im_rows),
        ],
        out_shape=[
            jax.ShapeDtypeStruct((N, n_cls), jnp.float32),
            jax.ShapeDtypeStruct((N, 8), jnp.float32),
            jax.ShapeDtypeStruct((N, 8), jnp.float32),
        ],
        compiler_params=pltpu.CompilerParams(
            dimension_semantics=("arbitrary",)),
    )
    return p0, p1


def kernel(feat1, feat2, msk, samp_bias1, samp_bias2, edge_index,
           W_mp, W_attn, b_attn, q_attn, W_fc, b_fc, M_disc):
    N, D = feat1.shape
    E = edge_index.shape[1]
    n_mp = W_mp.shape[0]
    n_cls = W_fc.shape[1]

    src = edge_index[0]
    dst = edge_index[1]

    sc_agg = _build_sc_agg(N, D, E)
    agg1, agg2, degp = sc_agg(feat1, feat2, src, dst)
    degp2 = degp.reshape(32, N).T

    mskc = msk.reshape(N, 1)
    bq = jnp.stack([b_attn, q_attn])
    p0, p1 = _build_tc_dense(N, D, n_mp, n_cls)
    w3, w3b, vv = p0(agg1, agg2, degp2, mskc, W_mp, W_attn, bq, M_disc)
    preds, sc1f, sc2f = p1(agg1, agg2, degp2, W_mp, w3, w3b, vv,
                           W_fc, b_fc.reshape(1, n_cls))

    weights = w3[:, 0]
    sc_1 = sc1f[:, 0][None, :] + samp_bias1
    sc_2 = sc2f[:, 0][None, :] + samp_bias2
    reg = jnp.concatenate([sc_1, sc_2], axis=1)
    return (preds, weights, reg)
